# baseline (speedup 1.0000x reference)
import functools

import jax
import jax.numpy as jnp
from jax import lax
from jax.experimental import pallas as pl
from jax.experimental.pallas import tpu as pltpu
from jax.experimental.pallas import tpu_sc as plsc

N = 10000
E = 320000
D_IN = 128
H = 64

NC = 2
NS = 16
NW = NC * NS
L = 16

CB = 128
C_TOT = 2560
E_PAD = C_TOT * CB
NCU = 1
Q0 = C_TOT // NS
N_ACC = 10240
CH_PER_SUB = N_ACC // CB // NS
N_SUB = N_ACC // NS
NBUF = 4

BLK = 1024
NBLK = N_ACC // BLK


@functools.lru_cache(maxsize=None)
def _make_sc_agg(D, with_count):
    mesh = plsc.VectorSubcoreMesh(core_axis_name="c", subcore_axis_name="s",
                                  num_cores=NCU)
    outs = [jax.ShapeDtypeStruct((NCU, N_ACC, D), jnp.float32)]
    if with_count:
        outs.append(jax.ShapeDtypeStruct((NCU, N_ACC), jnp.float32))
    scratch = [
        pltpu.VMEM((Q0, CB), jnp.int32),
        pltpu.VMEM((Q0, CB), jnp.int32),
    ]
    scratch += [pltpu.VMEM((CB, D), jnp.float32) for _ in range(NBUF)]
    scratch.append(pltpu.VMEM_SHARED((N_ACC, D), jnp.float32))
    if with_count:
        scratch.append(pltpu.VMEM((CB,), jnp.float32))
        scratch.append(pltpu.VMEM((N_SUB,), jnp.float32))
        scratch.append(pltpu.VMEM_SHARED((N_ACC,), jnp.float32))
    scratch += [pltpu.SemaphoreType.DMA for _ in range(2 * NBUF)]
    if with_count:
        scratch.append(pltpu.SemaphoreType.DMA)

    @functools.partial(
        pl.kernel,
        mesh=mesh,
        out_type=tuple(outs) if with_count else outs[0],
        scratch_types=scratch,
        compiler_params=pltpu.CompilerParams(use_tc_tiling_on_sc=False),
    )
    def agg(tab, packed, *rest):
        if with_count:
            (out, cnt_out, src_v, dst_v, *bufs, acc, ones_v, zc_v, cacc) = (
                rest[:-(2 * NBUF + 1)])
            sems = rest[-(2 * NBUF + 1):]
            gs, ss, csem = sems[:NBUF], sems[NBUF:2 * NBUF], sems[-1]
        else:
            (out, src_v, dst_v, *bufs, acc) = rest[:-(2 * NBUF)]
            sems = rest[-(2 * NBUF):]
            gs, ss = sems[:NBUF], sems[NBUF:]
            cnt_out = ones_v = zc_v = cacc = csem = None

        c = lax.axis_index("c")
        s = lax.axis_index("s")

        with jax.named_scope("agg_zero"):
            @pl.loop(0, CB)
            def _(i):
                @pl.loop(0, D // L)
                def _(k):
                    bufs[0][i, pl.ds(k * L, L)] = jnp.zeros((L,), jnp.float32)

            if with_count:
                @pl.loop(0, CB // L)
                def _(k):
                    ones_v[pl.ds(k * L, L)] = jnp.full((L,), 1.0, jnp.float32)

                @pl.loop(0, N_SUB // L)
                def _(k):
                    zc_v[pl.ds(k * L, L)] = jnp.zeros((L,), jnp.float32)

            for t in range(CH_PER_SUB):
                ch = s * CH_PER_SUB + t
                pltpu.sync_copy(bufs[0], acc.at[pl.ds(ch * CB, CB)])
            if with_count:
                pltpu.sync_copy(zc_v, cacc.at[pl.ds(s * N_SUB, N_SUB)])

        with jax.named_scope("agg_barrier1"):
            plsc.subcore_barrier()

        def run(nch, base):
            with jax.named_scope("agg_idx"):
                pltpu.sync_copy(packed.at[pl.ds(base, nch)],
                                src_v.at[pl.ds(0, nch)])

                @pl.loop(0, nch)
                def _(j):
                    @pl.loop(0, CB // L)
                    def _(k):
                        v = src_v[j, pl.ds(k * L, L)]
                        dst_v[j, pl.ds(k * L, L)] = jnp.bitwise_and(v, 16383)
                        src_v[j, pl.ds(k * L, L)] = jnp.right_shift(v, 14)

            pltpu.async_copy(tab.at[src_v.at[0]], bufs[0], gs[0])
            pltpu.async_copy(tab.at[src_v.at[1]], bufs[1], gs[1])

            def ring_body(g):
                for b in range(NBUF):
                    j = g + b
                    b2 = (b + 2) % NBUF

                    @pl.when(j >= 2)
                    def _():
                        pltpu.make_async_copy(
                            bufs[b2], acc.at[dst_v.at[j - 2]], ss[b2]).wait()

                    @pl.when(j + 2 < nch)
                    def _():
                        pltpu.async_copy(tab.at[src_v.at[j + 2]], bufs[b2],
                                         gs[b2])

                    pltpu.make_async_copy(tab.at[src_v.at[j]], bufs[b],
                                          gs[b]).wait()
                    pltpu.async_copy(bufs[b], acc.at[dst_v.at[j]], ss[b],
                                     add=True)
                    if with_count:
                        pltpu.async_copy(ones_v, cacc.at[dst_v.at[j]], csem,
                                         add=True)

            with jax.named_scope("agg_ring"):
                @pl.loop(0, nch, step=NBUF)
                def _(g):
                    ring_body(g)

            with jax.named_scope("agg_drain"):
                pltpu.make_async_copy(bufs[(nch - 2) % NBUF],
                                      acc.at[dst_v.at[nch - 2]],
                                      ss[(nch - 2) % NBUF]).wait()
                pltpu.make_async_copy(bufs[(nch - 1) % NBUF],
                                      acc.at[dst_v.at[nch - 1]],
                                      ss[(nch - 1) % NBUF]).wait()
                if with_count:
                    @pl.loop(0, nch)
                    def _(j):
                        pltpu.make_async_copy(ones_v, cacc.at[dst_v.at[j]],
                                              csem).wait()

        run(Q0, s * Q0)

        with jax.named_scope("agg_barrier2"):
            plsc.subcore_barrier()

        with jax.named_scope("agg_dump"):
            for t in range(CH_PER_SUB):
                ch = s * CH_PER_SUB + t
                pltpu.sync_copy(acc.at[pl.ds(ch * CB, CB)],
                                out.at[c, pl.ds(ch * CB, CB)])
            if with_count:
                pltpu.sync_copy(cacc.at[pl.ds(s * N_SUB, N_SUB)],
                                cnt_out.at[c, pl.ds(s * N_SUB, N_SUB)])

    return agg


def _tc_a_body(x_ref, w_ref, b_ref, o_ref):
    o_ref[...] = (jnp.dot(x_ref[...], w_ref[...],
                          preferred_element_type=jnp.float32) + b_ref[...])


def _tc_a(x, W1_l, b1):
    return pl.pallas_call(
        _tc_a_body,
        grid=(NBLK,),
        in_specs=[
            pl.BlockSpec((BLK, D_IN), lambda i: (i, 0)),
            pl.BlockSpec((D_IN, 2 * H), lambda i: (0, 0)),
            pl.BlockSpec((1, 2 * H), lambda i: (0, 0)),
        ],
        out_specs=pl.BlockSpec((BLK, 2 * H), lambda i: (i, 0)),
        out_shape=jax.ShapeDtypeStruct((N, 2 * H), jnp.float32),
    )(x, W1_l, b1.reshape(1, 2 * H))


def _rcnt(c_ref):
    cnt = jnp.maximum(c_ref[0], 1.0)
    return (1.0 / cnt)[:, None]


def _tc_b_body(p_ref, aa_ref, ab_ref, c_ref, w1ra_ref, w1rb_ref, w2_ref,
               b2_ref, o1_ref, o2_ref):
    rc = _rcnt(c_ref)
    m1a = aa_ref[0] * rc
    m1b = ab_ref[0] * rc
    h1 = jnp.maximum(
        p_ref[...]
        + jnp.dot(m1a, w1ra_ref[...], preferred_element_type=jnp.float32)
        + jnp.dot(m1b, w1rb_ref[...], preferred_element_type=jnp.float32),
        0.0)
    y2 = (jnp.dot(h1, w2_ref[...], preferred_element_type=jnp.float32)
          + b2_ref[...])
    o1_ref[...] = y2[:, :H]
    o2_ref[...] = y2[:, H:]


def _tc_b(p_l1, agg1a, agg1b, cntp, W1r_a, W1r_b, W2cat, b2cat):
    return pl.pallas_call(
        _tc_b_body,
        grid=(NBLK,),
        in_specs=[
            pl.BlockSpec((BLK, 2 * H), lambda i: (i, 0)),
            pl.BlockSpec((NCU, BLK, H), lambda i: (0, i, 0)),
            pl.BlockSpec((NCU, BLK, H), lambda i: (0, i, 0)),
            pl.BlockSpec((NCU, BLK), lambda i: (0, i)),
            pl.BlockSpec((H, 2 * H), lambda i: (0, 0)),
            pl.BlockSpec((H, 2 * H), lambda i: (0, 0)),
            pl.BlockSpec((2 * H, 2 * H), lambda i: (0, 0)),
            pl.BlockSpec((1, 2 * H), lambda i: (0, 0)),
        ],
        out_specs=[
            pl.BlockSpec((BLK, H), lambda i: (i, 0)),
            pl.BlockSpec((BLK, H), lambda i: (i, 0)),
        ],
        out_shape=[
            jax.ShapeDtypeStruct((N, H), jnp.float32),
            jax.ShapeDtypeStruct((N, H), jnp.float32),
        ],
    )(p_l1, agg1a, agg1b, cntp, W1r_a, W1r_b, W2cat, b2cat)


def _tc_c_body(p_ref, a_ref, c_ref, w_ref, b_ref, oh_ref, op_ref):
    m2 = a_ref[0] * _rcnt(c_ref)
    h2 = jnp.maximum(p_ref[...] + m2, 0.0)
    oh_ref[...] = h2
    op_ref[...] = (jnp.dot(h2, w_ref[...],
                           preferred_element_type=jnp.float32) + b_ref[...])


def _tc_c(p_l2b, agg2, cntp, Wmulv_l, bmulv):
    return pl.pallas_call(
        _tc_c_body,
        grid=(NBLK,),
        in_specs=[
            pl.BlockSpec((BLK, H), lambda i: (i, 0)),
            pl.BlockSpec((NCU, BLK, H), lambda i: (0, i, 0)),
            pl.BlockSpec((NCU, BLK), lambda i: (0, i)),
            pl.BlockSpec((H, 2 * H), lambda i: (0, 0)),
            pl.BlockSpec((1, 2 * H), lambda i: (0, 0)),
        ],
        out_specs=[
            pl.BlockSpec((BLK, H), lambda i: (i, 0)),
            pl.BlockSpec((BLK, 2 * H), lambda i: (i, 0)),
        ],
        out_shape=[
            jax.ShapeDtypeStruct((N, H), jnp.float32),
            jax.ShapeDtypeStruct((N, 2 * H), jnp.float32),
        ],
    )(p_l2b, agg2, cntp, Wmulv_l, bmulv)


def _tc_d_body(p_ref, a_ref, c_ref, w_ref, omu_ref, olv_ref):
    m3 = a_ref[0] * _rcnt(c_ref)
    res = p_ref[...] + jnp.dot(m3, w_ref[...],
                               preferred_element_type=jnp.float32)
    omu_ref[...] = res[:, :H]
    olv_ref[...] = res[:, H:]


def _tc_d(p_mulv, agg3, cntp, Wmulv_r):
    return pl.pallas_call(
        _tc_d_body,
        grid=(NBLK,),
        in_specs=[
            pl.BlockSpec((BLK, 2 * H), lambda i: (i, 0)),
            pl.BlockSpec((NCU, BLK, H), lambda i: (0, i, 0)),
            pl.BlockSpec((NCU, BLK), lambda i: (0, i)),
            pl.BlockSpec((H, 2 * H), lambda i: (0, 0)),
        ],
        out_specs=[
            pl.BlockSpec((BLK, H), lambda i: (i, 0)),
            pl.BlockSpec((BLK, H), lambda i: (i, 0)),
        ],
        out_shape=[
            jax.ShapeDtypeStruct((N, H), jnp.float32),
            jax.ShapeDtypeStruct((N, H), jnp.float32),
        ],
    )(p_mulv, agg3, cntp, Wmulv_r)


def kernel(x, edge_index, W1_l, b1, W1_r, W2_l, b2, W2_r,
           Wmu_l, bmu, Wmu_r, Wlv_l, blv, Wlv_r):
    src = edge_index[0].astype(jnp.int32)
    dst = edge_index[1].astype(jnp.int32)
    pad = E_PAD - E
    packed = jnp.concatenate(
        [src * 16384 + dst, jnp.full((pad,), N, jnp.int32)]).reshape(
            C_TOT, CB)

    W2cat = jnp.concatenate([W2_l, W2_r], axis=1)
    b2cat = jnp.concatenate([b2, jnp.zeros((H,), jnp.float32)]).reshape(1, 2 * H)
    Wmulv_l = jnp.concatenate([Wmu_l, Wlv_l], axis=1)
    bmulv = jnp.concatenate([bmu, blv]).reshape(1, 2 * H)
    Wmulv_r = jnp.concatenate([Wmu_r, Wlv_r], axis=1)

    agg1a, cntp = _make_sc_agg(H, True)(x[:, :H], packed)
    agg1b = _make_sc_agg(H, False)(x[:, H:], packed)
    p_l1 = _tc_a(x, W1_l, b1)
    p_l2b, t2 = _tc_b(p_l1, agg1a, agg1b, cntp, W1_r[:H], W1_r[H:],
                      W2cat, b2cat)
    agg2 = _make_sc_agg(H, False)(t2, packed)
    h2, p_mulv = _tc_c(p_l2b, agg2, cntp, Wmulv_l, bmulv)
    agg3 = _make_sc_agg(H, False)(h2, packed)
    mu, lv = _tc_d(p_mulv, agg3, cntp, Wmulv_r)
    return (mu, lv)


# speedup vs baseline: 3.3319x; 3.3319x over previous
import functools

import jax
import jax.numpy as jnp
from jax import lax
from jax.experimental import pallas as pl
from jax.experimental.pallas import tpu as pltpu
from jax.experimental.pallas import tpu_sc as plsc

N = 10000
E = 320000
D_IN = 128
H = 64

NC = 2
NS = 16
NW = NC * NS
L = 16

CB = 128
C_TOT = 2560
E_PAD = C_TOT * CB
NCU = 2
Q0 = C_TOT // NW
N_ACC = 10240
CH_PER_SUB = N_ACC // CB // NS
N_SUB = N_ACC // NS
NBUF = 4

BLK = 1024
NBLK = N_ACC // BLK


@functools.lru_cache(maxsize=None)
def _make_sc_agg(D, with_count):
    mesh = plsc.VectorSubcoreMesh(core_axis_name="c", subcore_axis_name="s",
                                  num_cores=NCU)
    outs = [jax.ShapeDtypeStruct((NCU, N_ACC, D), jnp.float32)]
    if with_count:
        outs.append(jax.ShapeDtypeStruct((NCU, N_ACC), jnp.float32))
    scratch = [
        pltpu.VMEM((Q0, CB), jnp.int32),
        pltpu.VMEM((Q0, CB), jnp.int32),
    ]
    scratch += [pltpu.VMEM((CB, D), jnp.float32) for _ in range(NBUF)]
    scratch.append(pltpu.VMEM_SHARED((N_ACC, D), jnp.float32))
    if with_count:
        scratch.append(pltpu.VMEM((CB,), jnp.float32))
        scratch.append(pltpu.VMEM((N_SUB,), jnp.float32))
        scratch.append(pltpu.VMEM_SHARED((N_ACC,), jnp.float32))
    scratch += [pltpu.SemaphoreType.DMA for _ in range(2 * NBUF)]
    if with_count:
        scratch.append(pltpu.SemaphoreType.DMA)

    @functools.partial(
        pl.kernel,
        mesh=mesh,
        out_type=tuple(outs) if with_count else outs[0],
        scratch_types=scratch,
        compiler_params=pltpu.CompilerParams(use_tc_tiling_on_sc=False),
    )
    def agg(tab, packed, *rest):
        if with_count:
            (out, cnt_out, src_v, dst_v, *bufs, acc, ones_v, zc_v, cacc) = (
                rest[:-(2 * NBUF + 1)])
            sems = rest[-(2 * NBUF + 1):]
            gs, ss, csem = sems[:NBUF], sems[NBUF:2 * NBUF], sems[-1]
        else:
            (out, src_v, dst_v, *bufs, acc) = rest[:-(2 * NBUF)]
            sems = rest[-(2 * NBUF):]
            gs, ss = sems[:NBUF], sems[NBUF:]
            cnt_out = ones_v = zc_v = cacc = csem = None

        c = lax.axis_index("c")
        s = lax.axis_index("s")

        with jax.named_scope("agg_zero"):
            @pl.loop(0, CB)
            def _(i):
                @pl.loop(0, D // L)
                def _(k):
                    bufs[0][i, pl.ds(k * L, L)] = jnp.zeros((L,), jnp.float32)

            if with_count:
                @pl.loop(0, CB // L)
                def _(k):
                    ones_v[pl.ds(k * L, L)] = jnp.full((L,), 1.0, jnp.float32)

                @pl.loop(0, N_SUB // L)
                def _(k):
                    zc_v[pl.ds(k * L, L)] = jnp.zeros((L,), jnp.float32)

            for t in range(CH_PER_SUB):
                ch = s * CH_PER_SUB + t
                pltpu.sync_copy(bufs[0], acc.at[pl.ds(ch * CB, CB)])
            if with_count:
                pltpu.sync_copy(zc_v, cacc.at[pl.ds(s * N_SUB, N_SUB)])

        with jax.named_scope("agg_barrier1"):
            plsc.subcore_barrier()

        def run(nch, base):
            with jax.named_scope("agg_idx"):
                pltpu.sync_copy(packed.at[pl.ds(base, nch)],
                                src_v.at[pl.ds(0, nch)])

                @pl.loop(0, nch)
                def _(j):
                    @pl.loop(0, CB // L)
                    def _(k):
                        v = src_v[j, pl.ds(k * L, L)]
                        dst_v[j, pl.ds(k * L, L)] = jnp.bitwise_and(v, 16383)
                        src_v[j, pl.ds(k * L, L)] = jnp.right_shift(v, 14)

            pltpu.async_copy(tab.at[src_v.at[0]], bufs[0], gs[0])
            pltpu.async_copy(tab.at[src_v.at[1]], bufs[1], gs[1])

            def ring_body(g):
                for b in range(NBUF):
                    j = g + b
                    b2 = (b + 2) % NBUF

                    @pl.when(j >= 2)
                    def _():
                        pltpu.make_async_copy(
                            bufs[b2], acc.at[dst_v.at[j - 2]], ss[b2]).wait()

                    @pl.when(j + 2 < nch)
                    def _():
                        pltpu.async_copy(tab.at[src_v.at[j + 2]], bufs[b2],
                                         gs[b2])

                    pltpu.make_async_copy(tab.at[src_v.at[j]], bufs[b],
                                          gs[b]).wait()
                    pltpu.async_copy(bufs[b], acc.at[dst_v.at[j]], ss[b],
                                     add=True)
                    if with_count:
                        pltpu.async_copy(ones_v, cacc.at[dst_v.at[j]], csem,
                                         add=True)

            with jax.named_scope("agg_ring"):
                @pl.loop(0, nch, step=NBUF)
                def _(g):
                    ring_body(g)

            with jax.named_scope("agg_drain"):
                pltpu.make_async_copy(bufs[(nch - 2) % NBUF],
                                      acc.at[dst_v.at[nch - 2]],
                                      ss[(nch - 2) % NBUF]).wait()
                pltpu.make_async_copy(bufs[(nch - 1) % NBUF],
                                      acc.at[dst_v.at[nch - 1]],
                                      ss[(nch - 1) % NBUF]).wait()
                if with_count:
                    @pl.loop(0, nch)
                    def _(j):
                        pltpu.make_async_copy(ones_v, cacc.at[dst_v.at[j]],
                                              csem).wait()

        run(Q0, (c * NS + s) * Q0)

        with jax.named_scope("agg_barrier2"):
            plsc.subcore_barrier()

        with jax.named_scope("agg_dump"):
            for t in range(CH_PER_SUB):
                ch = s * CH_PER_SUB + t
                pltpu.sync_copy(acc.at[pl.ds(ch * CB, CB)],
                                out.at[c, pl.ds(ch * CB, CB)])
            if with_count:
                pltpu.sync_copy(cacc.at[pl.ds(s * N_SUB, N_SUB)],
                                cnt_out.at[c, pl.ds(s * N_SUB, N_SUB)])

    return agg


def _tc_a_body(x_ref, w_ref, b_ref, o_ref):
    o_ref[...] = (jnp.dot(x_ref[...], w_ref[...],
                          preferred_element_type=jnp.float32) + b_ref[...])


def _tc_a(x, W1_l, b1):
    return pl.pallas_call(
        _tc_a_body,
        grid=(NBLK,),
        in_specs=[
            pl.BlockSpec((BLK, D_IN), lambda i: (i, 0)),
            pl.BlockSpec((D_IN, 2 * H), lambda i: (0, 0)),
            pl.BlockSpec((1, 2 * H), lambda i: (0, 0)),
        ],
        out_specs=pl.BlockSpec((BLK, 2 * H), lambda i: (i, 0)),
        out_shape=jax.ShapeDtypeStruct((N, 2 * H), jnp.float32),
    )(x, W1_l, b1.reshape(1, 2 * H))


def _rcnt(c_ref):
    cnt = jnp.maximum(c_ref[0] + c_ref[1], 1.0)
    return (1.0 / cnt)[:, None]


def _tc_b_body(p_ref, aa_ref, ab_ref, c_ref, w1ra_ref, w1rb_ref, w2_ref,
               b2_ref, o1_ref, o2_ref):
    rc = _rcnt(c_ref)
    m1a = (aa_ref[0] + aa_ref[1]) * rc
    m1b = (ab_ref[0] + ab_ref[1]) * rc
    h1 = jnp.maximum(
        p_ref[...]
        + jnp.dot(m1a, w1ra_ref[...], preferred_element_type=jnp.float32)
        + jnp.dot(m1b, w1rb_ref[...], preferred_element_type=jnp.float32),
        0.0)
    y2 = (jnp.dot(h1, w2_ref[...], preferred_element_type=jnp.float32)
          + b2_ref[...])
    o1_ref[...] = y2[:, :H]
    o2_ref[...] = y2[:, H:]


def _tc_b(p_l1, agg1a, agg1b, cntp, W1r_a, W1r_b, W2cat, b2cat):
    return pl.pallas_call(
        _tc_b_body,
        grid=(NBLK,),
        in_specs=[
            pl.BlockSpec((BLK, 2 * H), lambda i: (i, 0)),
            pl.BlockSpec((NCU, BLK, H), lambda i: (0, i, 0)),
            pl.BlockSpec((NCU, BLK, H), lambda i: (0, i, 0)),
            pl.BlockSpec((NCU, BLK), lambda i: (0, i)),
            pl.BlockSpec((H, 2 * H), lambda i: (0, 0)),
            pl.BlockSpec((H, 2 * H), lambda i: (0, 0)),
            pl.BlockSpec((2 * H, 2 * H), lambda i: (0, 0)),
            pl.BlockSpec((1, 2 * H), lambda i: (0, 0)),
        ],
        out_specs=[
            pl.BlockSpec((BLK, H), lambda i: (i, 0)),
            pl.BlockSpec((BLK, H), lambda i: (i, 0)),
        ],
        out_shape=[
            jax.ShapeDtypeStruct((N, H), jnp.float32),
            jax.ShapeDtypeStruct((N, H), jnp.float32),
        ],
    )(p_l1, agg1a, agg1b, cntp, W1r_a, W1r_b, W2cat, b2cat)


def _tc_c_body(p_ref, a_ref, c_ref, w_ref, b_ref, oh_ref, op_ref):
    m2 = (a_ref[0] + a_ref[1]) * _rcnt(c_ref)
    h2 = jnp.maximum(p_ref[...] + m2, 0.0)
    oh_ref[...] = h2
    op_ref[...] = (jnp.dot(h2, w_ref[...],
                           preferred_element_type=jnp.float32) + b_ref[...])


def _tc_c(p_l2b, agg2, cntp, Wmulv_l, bmulv):
    return pl.pallas_call(
        _tc_c_body,
        grid=(NBLK,),
        in_specs=[
            pl.BlockSpec((BLK, H), lambda i: (i, 0)),
            pl.BlockSpec((NCU, BLK, H), lambda i: (0, i, 0)),
            pl.BlockSpec((NCU, BLK), lambda i: (0, i)),
            pl.BlockSpec((H, 2 * H), lambda i: (0, 0)),
            pl.BlockSpec((1, 2 * H), lambda i: (0, 0)),
        ],
        out_specs=[
            pl.BlockSpec((BLK, H), lambda i: (i, 0)),
            pl.BlockSpec((BLK, 2 * H), lambda i: (i, 0)),
        ],
        out_shape=[
            jax.ShapeDtypeStruct((N, H), jnp.float32),
            jax.ShapeDtypeStruct((N, 2 * H), jnp.float32),
        ],
    )(p_l2b, agg2, cntp, Wmulv_l, bmulv)


def _tc_d_body(p_ref, a_ref, c_ref, w_ref, omu_ref, olv_ref):
    m3 = (a_ref[0] + a_ref[1]) * _rcnt(c_ref)
    res = p_ref[...] + jnp.dot(m3, w_ref[...],
                               preferred_element_type=jnp.float32)
    omu_ref[...] = res[:, :H]
    olv_ref[...] = res[:, H:]


def _tc_d(p_mulv, agg3, cntp, Wmulv_r):
    return pl.pallas_call(
        _tc_d_body,
        grid=(NBLK,),
        in_specs=[
            pl.BlockSpec((BLK, 2 * H), lambda i: (i, 0)),
            pl.BlockSpec((NCU, BLK, H), lambda i: (0, i, 0)),
            pl.BlockSpec((NCU, BLK), lambda i: (0, i)),
            pl.BlockSpec((H, 2 * H), lambda i: (0, 0)),
        ],
        out_specs=[
            pl.BlockSpec((BLK, H), lambda i: (i, 0)),
            pl.BlockSpec((BLK, H), lambda i: (i, 0)),
        ],
        out_shape=[
            jax.ShapeDtypeStruct((N, H), jnp.float32),
            jax.ShapeDtypeStruct((N, H), jnp.float32),
        ],
    )(p_mulv, agg3, cntp, Wmulv_r)


def kernel(x, edge_index, W1_l, b1, W1_r, W2_l, b2, W2_r,
           Wmu_l, bmu, Wmu_r, Wlv_l, blv, Wlv_r):
    src = edge_index[0].astype(jnp.int32)
    dst = edge_index[1].astype(jnp.int32)
    pad = E_PAD - E
    it = jnp.arange(pad, dtype=jnp.int32)
    pad_packed = (it % N) * 16384 + (N + it % (N_ACC - N))
    packed = jnp.concatenate([src * 16384 + dst, pad_packed]).reshape(
        C_TOT, CB)

    W2cat = jnp.concatenate([W2_l, W2_r], axis=1)
    b2cat = jnp.concatenate([b2, jnp.zeros((H,), jnp.float32)]).reshape(1, 2 * H)
    Wmulv_l = jnp.concatenate([Wmu_l, Wlv_l], axis=1)
    bmulv = jnp.concatenate([bmu, blv]).reshape(1, 2 * H)
    Wmulv_r = jnp.concatenate([Wmu_r, Wlv_r], axis=1)

    agg1a, cntp = _make_sc_agg(H, True)(x[:, :H], packed)
    agg1b = _make_sc_agg(H, False)(x[:, H:], packed)
    p_l1 = _tc_a(x, W1_l, b1)
    p_l2b, t2 = _tc_b(p_l1, agg1a, agg1b, cntp, W1_r[:H], W1_r[H:],
                      W2cat, b2cat)
    agg2 = _make_sc_agg(H, False)(t2, packed)
    h2, p_mulv = _tc_c(p_l2b, agg2, cntp, Wmulv_l, bmulv)
    agg3 = _make_sc_agg(H, False)(h2, packed)
    mu, lv = _tc_d(p_mulv, agg3, cntp, Wmulv_r)
    return (mu, lv)


# speedup vs baseline: 3.4485x; 1.0350x over previous
import functools

import jax
import jax.numpy as jnp
from jax import lax
from jax.experimental import pallas as pl
from jax.experimental.pallas import tpu as pltpu
from jax.experimental.pallas import tpu_sc as plsc

N = 10000
E = 320000
D_IN = 128
H = 64

NC = 2
NS = 16
NW = NC * NS
L = 16

CB = 128
C_TOT = 2560
E_PAD = C_TOT * CB
NCU = 2
Q0 = C_TOT // NW
N_ACC = 10240
CH_PER_SUB = N_ACC // CB // NS
N_SUB = N_ACC // NS
NBUF = 4

BLK = 1024
NBLK = N_ACC // BLK


@functools.lru_cache(maxsize=None)
def _make_sc_agg(D, with_count, two_tables=False):
    mesh = plsc.VectorSubcoreMesh(core_axis_name="c", subcore_axis_name="s",
                                  num_cores=NCU)
    QMAX = C_TOT // NS if two_tables else Q0
    outs = [jax.ShapeDtypeStruct((NCU, N_ACC, D), jnp.float32)]
    if with_count:
        outs.append(jax.ShapeDtypeStruct((NCU, N_ACC), jnp.float32))
    scratch = [
        pltpu.VMEM((QMAX, CB), jnp.int32),
        pltpu.VMEM((QMAX, CB), jnp.int32),
    ]
    scratch += [pltpu.VMEM((CB, D), jnp.float32) for _ in range(NBUF)]
    scratch.append(pltpu.VMEM_SHARED((N_ACC, D), jnp.float32))
    if with_count:
        scratch.append(pltpu.VMEM((CB,), jnp.float32))
        scratch.append(pltpu.VMEM((N_SUB,), jnp.float32))
        scratch.append(pltpu.VMEM_SHARED((N_ACC,), jnp.float32))
    scratch += [pltpu.SemaphoreType.DMA for _ in range(2 * NBUF)]
    if with_count:
        scratch.append(pltpu.SemaphoreType.DMA)

    @functools.partial(
        pl.kernel,
        mesh=mesh,
        out_type=tuple(outs) if with_count else outs[0],
        scratch_types=scratch,
        compiler_params=pltpu.CompilerParams(use_tc_tiling_on_sc=False),
    )
    def agg(*args):
        if two_tables:
            tab_a, tab_b, srcs, dsts, *rest = args
        else:
            tab, srcs, dsts, *rest = args
        if with_count:
            (out, cnt_out, src_v, dst_v, *bufs, acc, ones_v, zc_v, cacc) = (
                rest[:-(2 * NBUF + 1)])
            sems = rest[-(2 * NBUF + 1):]
            gs, ss, csem = sems[:NBUF], sems[NBUF:2 * NBUF], sems[-1]
        else:
            (out, src_v, dst_v, *bufs, acc) = rest[:-(2 * NBUF)]
            sems = rest[-(2 * NBUF):]
            gs, ss = sems[:NBUF], sems[NBUF:]
            cnt_out = ones_v = zc_v = cacc = csem = None

        c = lax.axis_index("c")
        s = lax.axis_index("s")

        with jax.named_scope("agg_zero"):
            @pl.loop(0, CB)
            def _(i):
                @pl.loop(0, D // L)
                def _(k):
                    bufs[0][i, pl.ds(k * L, L)] = jnp.zeros((L,), jnp.float32)

            if with_count:
                @pl.loop(0, CB // L)
                def _(k):
                    ones_v[pl.ds(k * L, L)] = jnp.full((L,), 1.0, jnp.float32)

                @pl.loop(0, N_SUB // L)
                def _(k):
                    zc_v[pl.ds(k * L, L)] = jnp.zeros((L,), jnp.float32)

            for t in range(CH_PER_SUB):
                ch = s * CH_PER_SUB + t
                pltpu.sync_copy(bufs[0], acc.at[pl.ds(ch * CB, CB)])
            if with_count:
                pltpu.sync_copy(zc_v, cacc.at[pl.ds(s * N_SUB, N_SUB)])

        with jax.named_scope("agg_barrier1"):
            plsc.subcore_barrier()

        def run(tab, nch, base, do_count):
            with jax.named_scope("agg_idx"):
                pltpu.sync_copy(srcs.at[pl.ds(base, nch)],
                                src_v.at[pl.ds(0, nch)])
                pltpu.sync_copy(dsts.at[pl.ds(base, nch)],
                                dst_v.at[pl.ds(0, nch)])

            pltpu.async_copy(tab.at[src_v.at[0]], bufs[0], gs[0])
            pltpu.async_copy(tab.at[src_v.at[1]], bufs[1], gs[1])

            def ring_body(g):
                for b in range(NBUF):
                    j = g + b
                    b2 = (b + 2) % NBUF

                    @pl.when(j >= 2)
                    def _():
                        pltpu.make_async_copy(
                            bufs[b2], acc.at[dst_v.at[j - 2]], ss[b2]).wait()

                    @pl.when(j + 2 < nch)
                    def _():
                        pltpu.async_copy(tab.at[src_v.at[j + 2]], bufs[b2],
                                         gs[b2])

                    pltpu.make_async_copy(tab.at[src_v.at[j]], bufs[b],
                                          gs[b]).wait()
                    pltpu.async_copy(bufs[b], acc.at[dst_v.at[j]], ss[b],
                                     add=True)
                    if do_count:
                        pltpu.async_copy(ones_v, cacc.at[dst_v.at[j]], csem,
                                         add=True)

            with jax.named_scope("agg_ring"):
                @pl.loop(0, nch, step=NBUF)
                def _(g):
                    ring_body(g)

            with jax.named_scope("agg_drain"):
                pltpu.make_async_copy(bufs[(nch - 2) % NBUF],
                                      acc.at[dst_v.at[nch - 2]],
                                      ss[(nch - 2) % NBUF]).wait()
                pltpu.make_async_copy(bufs[(nch - 1) % NBUF],
                                      acc.at[dst_v.at[nch - 1]],
                                      ss[(nch - 1) % NBUF]).wait()
                if do_count:
                    @pl.loop(0, nch)
                    def _(j):
                        pltpu.make_async_copy(ones_v, cacc.at[dst_v.at[j]],
                                              csem).wait()

        if two_tables:
            @pl.when(c == 0)
            def _():
                run(tab_a, QMAX, s * QMAX, with_count)

            @pl.when(c == 1)
            def _():
                run(tab_b, QMAX, s * QMAX, False)
        else:
            run(tab, Q0, (c * NS + s) * Q0, with_count)

        with jax.named_scope("agg_barrier2"):
            plsc.subcore_barrier()

        with jax.named_scope("agg_dump"):
            for t in range(CH_PER_SUB):
                ch = s * CH_PER_SUB + t
                pltpu.sync_copy(acc.at[pl.ds(ch * CB, CB)],
                                out.at[c, pl.ds(ch * CB, CB)])
            if with_count:
                pltpu.sync_copy(cacc.at[pl.ds(s * N_SUB, N_SUB)],
                                cnt_out.at[c, pl.ds(s * N_SUB, N_SUB)])

    return agg


def _tc_a_body(x_ref, w_ref, b_ref, o_ref):
    o_ref[...] = (jnp.dot(x_ref[...], w_ref[...],
                          preferred_element_type=jnp.float32) + b_ref[...])


def _tc_a(x, W1_l, b1):
    return pl.pallas_call(
        _tc_a_body,
        grid=(NBLK,),
        in_specs=[
            pl.BlockSpec((BLK, D_IN), lambda i: (i, 0)),
            pl.BlockSpec((D_IN, 2 * H), lambda i: (0, 0)),
            pl.BlockSpec((1, 2 * H), lambda i: (0, 0)),
        ],
        out_specs=pl.BlockSpec((BLK, 2 * H), lambda i: (i, 0)),
        out_shape=jax.ShapeDtypeStruct((N, 2 * H), jnp.float32),
    )(x, W1_l, b1.reshape(1, 2 * H))


def _rcnt(c_ref):
    cnt = jnp.maximum(c_ref[0] + c_ref[1], 1.0)
    return (1.0 / cnt)[:, None]


def _tc_b_body(p_ref, a_ref, c_ref, w1ra_ref, w1rb_ref, w2_ref,
               b2_ref, o1_ref, o2_ref):
    rc = _rcnt(c_ref)
    m1a = a_ref[0] * rc
    m1b = a_ref[1] * rc
    h1 = jnp.maximum(
        p_ref[...]
        + jnp.dot(m1a, w1ra_ref[...], preferred_element_type=jnp.float32)
        + jnp.dot(m1b, w1rb_ref[...], preferred_element_type=jnp.float32),
        0.0)
    y2 = (jnp.dot(h1, w2_ref[...], preferred_element_type=jnp.float32)
          + b2_ref[...])
    o1_ref[...] = y2[:, :H]
    o2_ref[...] = y2[:, H:]


def _tc_b(p_l1, agg1, cntp, W1r_a, W1r_b, W2cat, b2cat):
    return pl.pallas_call(
        _tc_b_body,
        grid=(NBLK,),
        in_specs=[
            pl.BlockSpec((BLK, 2 * H), lambda i: (i, 0)),
            pl.BlockSpec((NCU, BLK, H), lambda i: (0, i, 0)),
            pl.BlockSpec((NCU, BLK), lambda i: (0, i)),
            pl.BlockSpec((H, 2 * H), lambda i: (0, 0)),
            pl.BlockSpec((H, 2 * H), lambda i: (0, 0)),
            pl.BlockSpec((2 * H, 2 * H), lambda i: (0, 0)),
            pl.BlockSpec((1, 2 * H), lambda i: (0, 0)),
        ],
        out_specs=[
            pl.BlockSpec((BLK, H), lambda i: (i, 0)),
            pl.BlockSpec((BLK, H), lambda i: (i, 0)),
        ],
        out_shape=[
            jax.ShapeDtypeStruct((N, H), jnp.float32),
            jax.ShapeDtypeStruct((N, H), jnp.float32),
        ],
    )(p_l1, agg1, cntp, W1r_a, W1r_b, W2cat, b2cat)


def _tc_c_body(p_ref, a_ref, c_ref, w_ref, b_ref, oh_ref, op_ref):
    m2 = (a_ref[0] + a_ref[1]) * _rcnt(c_ref)
    h2 = jnp.maximum(p_ref[...] + m2, 0.0)
    oh_ref[...] = h2
    op_ref[...] = (jnp.dot(h2, w_ref[...],
                           preferred_element_type=jnp.float32) + b_ref[...])


def _tc_c(p_l2b, agg2, cntp, Wmulv_l, bmulv):
    return pl.pallas_call(
        _tc_c_body,
        grid=(NBLK,),
        in_specs=[
            pl.BlockSpec((BLK, H), lambda i: (i, 0)),
            pl.BlockSpec((NCU, BLK, H), lambda i: (0, i, 0)),
            pl.BlockSpec((NCU, BLK), lambda i: (0, i)),
            pl.BlockSpec((H, 2 * H), lambda i: (0, 0)),
            pl.BlockSpec((1, 2 * H), lambda i: (0, 0)),
        ],
        out_specs=[
            pl.BlockSpec((BLK, H), lambda i: (i, 0)),
            pl.BlockSpec((BLK, 2 * H), lambda i: (i, 0)),
        ],
        out_shape=[
            jax.ShapeDtypeStruct((N, H), jnp.float32),
            jax.ShapeDtypeStruct((N, 2 * H), jnp.float32),
        ],
    )(p_l2b, agg2, cntp, Wmulv_l, bmulv)


def _tc_d_body(p_ref, a_ref, c_ref, w_ref, omu_ref, olv_ref):
    m3 = (a_ref[0] + a_ref[1]) * _rcnt(c_ref)
    res = p_ref[...] + jnp.dot(m3, w_ref[...],
                               preferred_element_type=jnp.float32)
    omu_ref[...] = res[:, :H]
    olv_ref[...] = res[:, H:]


def _tc_d(p_mulv, agg3, cntp, Wmulv_r):
    return pl.pallas_call(
        _tc_d_body,
        grid=(NBLK,),
        in_specs=[
            pl.BlockSpec((BLK, 2 * H), lambda i: (i, 0)),
            pl.BlockSpec((NCU, BLK, H), lambda i: (0, i, 0)),
            pl.BlockSpec((NCU, BLK), lambda i: (0, i)),
            pl.BlockSpec((H, 2 * H), lambda i: (0, 0)),
        ],
        out_specs=[
            pl.BlockSpec((BLK, H), lambda i: (i, 0)),
            pl.BlockSpec((BLK, H), lambda i: (i, 0)),
        ],
        out_shape=[
            jax.ShapeDtypeStruct((N, H), jnp.float32),
            jax.ShapeDtypeStruct((N, H), jnp.float32),
        ],
    )(p_mulv, agg3, cntp, Wmulv_r)


def kernel(x, edge_index, W1_l, b1, W1_r, W2_l, b2, W2_r,
           Wmu_l, bmu, Wmu_r, Wlv_l, blv, Wlv_r):
    src = edge_index[0].astype(jnp.int32)
    dst = edge_index[1].astype(jnp.int32)
    pad = E_PAD - E
    it = jnp.arange(pad, dtype=jnp.int32)
    srcs = jnp.concatenate([src, it % N]).reshape(C_TOT, CB)
    dsts = jnp.concatenate([dst, N + it % (N_ACC - N)]).reshape(C_TOT, CB)

    W2cat = jnp.concatenate([W2_l, W2_r], axis=1)
    b2cat = jnp.concatenate([b2, jnp.zeros((H,), jnp.float32)]).reshape(1, 2 * H)
    Wmulv_l = jnp.concatenate([Wmu_l, Wlv_l], axis=1)
    bmulv = jnp.concatenate([bmu, blv]).reshape(1, 2 * H)
    Wmulv_r = jnp.concatenate([Wmu_r, Wlv_r], axis=1)

    agg1, cntp = _make_sc_agg(H, True, True)(x[:, :H], x[:, H:], srcs, dsts)
    p_l1 = _tc_a(x, W1_l, b1)
    p_l2b, t2 = _tc_b(p_l1, agg1, cntp, W1_r[:H], W1_r[H:], W2cat, b2cat)
    agg2 = _make_sc_agg(H, False)(t2, srcs, dsts)
    h2, p_mulv = _tc_c(p_l2b, agg2, cntp, Wmulv_l, bmulv)
    agg3 = _make_sc_agg(H, False)(h2, srcs, dsts)
    mu, lv = _tc_d(p_mulv, agg3, cntp, Wmulv_r)
    return (mu, lv)


# speedup vs baseline: 3.6155x; 1.0484x over previous
import functools

import jax
import jax.numpy as jnp
from jax import lax
from jax.experimental import pallas as pl
from jax.experimental.pallas import tpu as pltpu
from jax.experimental.pallas import tpu_sc as plsc

N = 10000
E = 320000
D_IN = 128
H = 64

NC = 2
NS = 16
NW = NC * NS
L = 16

CB = 128
C_TOT = 2560
E_PAD = C_TOT * CB
NCU = 2
Q0 = C_TOT // NW
N_ACC = 10240
CH_PER_SUB = N_ACC // CB // NS
N_SUB = N_ACC // NS
NBUF = 4
PF = 2

BLK = 1024
NBLK = N_ACC // BLK


@functools.lru_cache(maxsize=None)
def _make_sc_agg(D, with_count, two_tables=False):
    mesh = plsc.VectorSubcoreMesh(core_axis_name="c", subcore_axis_name="s",
                                  num_cores=NCU)
    QMAX = C_TOT // NS if two_tables else Q0
    outs = [jax.ShapeDtypeStruct((NCU, N_ACC, D), jnp.float32)]
    if with_count:
        outs.append(jax.ShapeDtypeStruct((NCU, N_ACC), jnp.float32))
    scratch = [
        pltpu.VMEM((QMAX, CB), jnp.int32),
        pltpu.VMEM((QMAX, CB), jnp.int32),
    ]
    scratch += [pltpu.VMEM((CB, D), jnp.float32) for _ in range(NBUF)]
    scratch.append(pltpu.VMEM_SHARED((N_ACC, D), jnp.float32))
    if with_count:
        scratch.append(pltpu.VMEM((CB,), jnp.float32))
        scratch.append(pltpu.VMEM((N_SUB,), jnp.float32))
        scratch.append(pltpu.VMEM_SHARED((N_ACC,), jnp.float32))
    scratch += [pltpu.SemaphoreType.DMA for _ in range(2 * NBUF)]
    if with_count:
        scratch.append(pltpu.SemaphoreType.DMA)

    @functools.partial(
        pl.kernel,
        mesh=mesh,
        out_type=tuple(outs) if with_count else outs[0],
        scratch_types=scratch,
        compiler_params=pltpu.CompilerParams(use_tc_tiling_on_sc=False),
    )
    def agg(*args):
        tab, srcs, dsts, *rest = args
        if with_count:
            (out, cnt_out, src_v, dst_v, *bufs, acc, ones_v, zc_v, cacc) = (
                rest[:-(2 * NBUF + 1)])
            sems = rest[-(2 * NBUF + 1):]
            gs, ss, csem = sems[:NBUF], sems[NBUF:2 * NBUF], sems[-1]
        else:
            (out, src_v, dst_v, *bufs, acc) = rest[:-(2 * NBUF)]
            sems = rest[-(2 * NBUF):]
            gs, ss = sems[:NBUF], sems[NBUF:]
            cnt_out = ones_v = zc_v = cacc = csem = None

        c = lax.axis_index("c")
        s = lax.axis_index("s")

        with jax.named_scope("agg_zero"):
            @pl.loop(0, CB)
            def _(i):
                @pl.loop(0, D // L)
                def _(k):
                    bufs[0][i, pl.ds(k * L, L)] = jnp.zeros((L,), jnp.float32)

            if with_count:
                @pl.loop(0, CB // L)
                def _(k):
                    ones_v[pl.ds(k * L, L)] = jnp.full((L,), 1.0, jnp.float32)

                @pl.loop(0, N_SUB // L)
                def _(k):
                    zc_v[pl.ds(k * L, L)] = jnp.zeros((L,), jnp.float32)

            for t in range(CH_PER_SUB):
                ch = s * CH_PER_SUB + t
                pltpu.sync_copy(bufs[0], acc.at[pl.ds(ch * CB, CB)])
            if with_count:
                pltpu.sync_copy(zc_v, cacc.at[pl.ds(s * N_SUB, N_SUB)])

        with jax.named_scope("agg_barrier1"):
            plsc.subcore_barrier()

        def run(nch, base, do_count):
            with jax.named_scope("agg_idx"):
                pltpu.sync_copy(srcs.at[pl.ds(base, nch)],
                                src_v.at[pl.ds(0, nch)])
                pltpu.sync_copy(dsts.at[pl.ds(base, nch)],
                                dst_v.at[pl.ds(0, nch)])
                if two_tables:
                    @pl.loop(0, nch)
                    def _(j):
                        @pl.loop(0, CB // L)
                        def _(k):
                            v = src_v[j, pl.ds(k * L, L)]
                            src_v[j, pl.ds(k * L, L)] = v + v + c

            for k in range(PF):
                pltpu.async_copy(tab.at[src_v.at[k]], bufs[k], gs[k])

            def ring_body(g):
                for b in range(NBUF):
                    j = g + b
                    b2 = (b + PF) % NBUF

                    @pl.when(j >= NBUF - PF)
                    def _():
                        pltpu.make_async_copy(
                            bufs[b2], acc.at[dst_v.at[j - (NBUF - PF)]],
                            ss[b2]).wait()

                    @pl.when(j + PF < nch)
                    def _():
                        pltpu.async_copy(tab.at[src_v.at[j + PF]], bufs[b2],
                                         gs[b2])

                    pltpu.make_async_copy(tab.at[src_v.at[j]], bufs[b],
                                          gs[b]).wait()
                    pltpu.async_copy(bufs[b], acc.at[dst_v.at[j]], ss[b],
                                     add=True)
                    if do_count:
                        pltpu.async_copy(ones_v, cacc.at[dst_v.at[j]], csem,
                                         add=True)

            with jax.named_scope("agg_ring"):
                @pl.loop(0, nch, step=NBUF)
                def _(g):
                    ring_body(g)

            with jax.named_scope("agg_drain"):
                for k in range(nch - NBUF + PF, nch):
                    pltpu.make_async_copy(bufs[k % NBUF],
                                          acc.at[dst_v.at[k]],
                                          ss[k % NBUF]).wait()
                if do_count:
                    @pl.loop(0, nch)
                    def _(j):
                        pltpu.make_async_copy(ones_v, cacc.at[dst_v.at[j]],
                                              csem).wait()

        if two_tables:
            if with_count:
                @pl.when(c == 0)
                def _():
                    run(QMAX, s * QMAX, True)

                @pl.when(c == 1)
                def _():
                    run(QMAX, s * QMAX, False)
            else:
                run(QMAX, s * QMAX, False)
        else:
            run(Q0, (c * NS + s) * Q0, with_count)

        with jax.named_scope("agg_barrier2"):
            plsc.subcore_barrier()

        with jax.named_scope("agg_dump"):
            for t in range(CH_PER_SUB):
                ch = s * CH_PER_SUB + t
                pltpu.sync_copy(acc.at[pl.ds(ch * CB, CB)],
                                out.at[c, pl.ds(ch * CB, CB)])
            if with_count:
                pltpu.sync_copy(cacc.at[pl.ds(s * N_SUB, N_SUB)],
                                cnt_out.at[c, pl.ds(s * N_SUB, N_SUB)])

    return agg


def _tc_a_body(x_ref, w_ref, b_ref, o_ref):
    o_ref[...] = (jnp.dot(x_ref[...], w_ref[...],
                          preferred_element_type=jnp.float32) + b_ref[...])


def _tc_a(x, W1_l, b1):
    return pl.pallas_call(
        _tc_a_body,
        grid=(NBLK,),
        in_specs=[
            pl.BlockSpec((BLK, D_IN), lambda i: (i, 0)),
            pl.BlockSpec((D_IN, 2 * H), lambda i: (0, 0)),
            pl.BlockSpec((1, 2 * H), lambda i: (0, 0)),
        ],
        out_specs=pl.BlockSpec((BLK, 2 * H), lambda i: (i, 0)),
        out_shape=jax.ShapeDtypeStruct((N, 2 * H), jnp.float32),
    )(x, W1_l, b1.reshape(1, 2 * H))


def _rcnt(c_ref):
    cnt = jnp.maximum(c_ref[0] + c_ref[1], 1.0)
    return (1.0 / cnt)[:, None]


def _tc_b_body(p_ref, a_ref, c_ref, w1ra_ref, w1rb_ref, w2_ref,
               b2_ref, o1_ref, o2_ref):
    rc = _rcnt(c_ref)
    m1a = a_ref[0] * rc
    m1b = a_ref[1] * rc
    h1 = jnp.maximum(
        p_ref[...]
        + jnp.dot(m1a, w1ra_ref[...], preferred_element_type=jnp.float32)
        + jnp.dot(m1b, w1rb_ref[...], preferred_element_type=jnp.float32),
        0.0)
    y2 = (jnp.dot(h1, w2_ref[...], preferred_element_type=jnp.float32)
          + b2_ref[...])
    o1_ref[...] = y2[:, :H]
    o2_ref[...] = y2[:, H:]


def _tc_b(p_l1, agg1, cntp, W1r_a, W1r_b, W2cat, b2cat):
    return pl.pallas_call(
        _tc_b_body,
        grid=(NBLK,),
        in_specs=[
            pl.BlockSpec((BLK, 2 * H), lambda i: (i, 0)),
            pl.BlockSpec((NCU, BLK, H), lambda i: (0, i, 0)),
            pl.BlockSpec((NCU, BLK), lambda i: (0, i)),
            pl.BlockSpec((H, 2 * H), lambda i: (0, 0)),
            pl.BlockSpec((H, 2 * H), lambda i: (0, 0)),
            pl.BlockSpec((2 * H, 2 * H), lambda i: (0, 0)),
            pl.BlockSpec((1, 2 * H), lambda i: (0, 0)),
        ],
        out_specs=[
            pl.BlockSpec((BLK, H), lambda i: (i, 0)),
            pl.BlockSpec((BLK, H), lambda i: (i, 0)),
        ],
        out_shape=[
            jax.ShapeDtypeStruct((N, H), jnp.float32),
            jax.ShapeDtypeStruct((N, H), jnp.float32),
        ],
    )(p_l1, agg1, cntp, W1r_a, W1r_b, W2cat, b2cat)


def _tc_c_body(p_ref, a_ref, c_ref, w_ref, b_ref, oh_ref, op_ref):
    m2 = (a_ref[0] + a_ref[1]) * _rcnt(c_ref)
    h2 = jnp.maximum(p_ref[...] + m2, 0.0)
    oh_ref[...] = h2
    op_ref[...] = (jnp.dot(h2, w_ref[...],
                           preferred_element_type=jnp.float32) + b_ref[...])


def _tc_c(p_l2b, agg2, cntp, Wmulv_l, bmulv):
    return pl.pallas_call(
        _tc_c_body,
        grid=(NBLK,),
        in_specs=[
            pl.BlockSpec((BLK, H), lambda i: (i, 0)),
            pl.BlockSpec((NCU, BLK, H), lambda i: (0, i, 0)),
            pl.BlockSpec((NCU, BLK), lambda i: (0, i)),
            pl.BlockSpec((H, 2 * H), lambda i: (0, 0)),
            pl.BlockSpec((1, 2 * H), lambda i: (0, 0)),
        ],
        out_specs=[
            pl.BlockSpec((BLK, H), lambda i: (i, 0)),
            pl.BlockSpec((BLK, 2 * H), lambda i: (i, 0)),
        ],
        out_shape=[
            jax.ShapeDtypeStruct((N, H), jnp.float32),
            jax.ShapeDtypeStruct((N, 2 * H), jnp.float32),
        ],
    )(p_l2b, agg2, cntp, Wmulv_l, bmulv)


def _tc_d_body(p_ref, a_ref, c_ref, w_ref, omu_ref, olv_ref):
    m3 = (a_ref[0] + a_ref[1]) * _rcnt(c_ref)
    res = p_ref[...] + jnp.dot(m3, w_ref[...],
                               preferred_element_type=jnp.float32)
    omu_ref[...] = res[:, :H]
    olv_ref[...] = res[:, H:]


def _tc_d(p_mulv, agg3, cntp, Wmulv_r):
    return pl.pallas_call(
        _tc_d_body,
        grid=(NBLK,),
        in_specs=[
            pl.BlockSpec((BLK, 2 * H), lambda i: (i, 0)),
            pl.BlockSpec((NCU, BLK, H), lambda i: (0, i, 0)),
            pl.BlockSpec((NCU, BLK), lambda i: (0, i)),
            pl.BlockSpec((H, 2 * H), lambda i: (0, 0)),
        ],
        out_specs=[
            pl.BlockSpec((BLK, H), lambda i: (i, 0)),
            pl.BlockSpec((BLK, H), lambda i: (i, 0)),
        ],
        out_shape=[
            jax.ShapeDtypeStruct((N, H), jnp.float32),
            jax.ShapeDtypeStruct((N, H), jnp.float32),
        ],
    )(p_mulv, agg3, cntp, Wmulv_r)


def kernel(x, edge_index, W1_l, b1, W1_r, W2_l, b2, W2_r,
           Wmu_l, bmu, Wmu_r, Wlv_l, blv, Wlv_r):
    src = edge_index[0].astype(jnp.int32)
    dst = edge_index[1].astype(jnp.int32)
    pad = E_PAD - E
    it = jnp.arange(pad, dtype=jnp.int32)
    srcs = jnp.concatenate([src, it % N]).reshape(C_TOT, CB)
    dsts = jnp.concatenate([dst, N + it % (N_ACC - N)]).reshape(C_TOT, CB)

    W2cat = jnp.concatenate([W2_l, W2_r], axis=1)
    b2cat = jnp.concatenate([b2, jnp.zeros((H,), jnp.float32)]).reshape(1, 2 * H)
    Wmulv_l = jnp.concatenate([Wmu_l, Wlv_l], axis=1)
    bmulv = jnp.concatenate([bmu, blv]).reshape(1, 2 * H)
    Wmulv_r = jnp.concatenate([Wmu_r, Wlv_r], axis=1)

    agg1, cntp = _make_sc_agg(H, True, True)(x.reshape(2 * N, H), srcs, dsts)
    p_l1 = _tc_a(x, W1_l, b1)
    p_l2b, t2 = _tc_b(p_l1, agg1, cntp, W1_r[:H], W1_r[H:], W2cat, b2cat)
    agg2 = _make_sc_agg(H, False)(t2, srcs, dsts)
    h2, p_mulv = _tc_c(p_l2b, agg2, cntp, Wmulv_l, bmulv)
    agg3 = _make_sc_agg(H, False)(h2, srcs, dsts)
    mu, lv = _tc_d(p_mulv, agg3, cntp, Wmulv_r)
    return (mu, lv)


# speedup vs baseline: 3.6942x; 1.0218x over previous
import functools

import jax
import jax.numpy as jnp
from jax import lax
from jax.experimental import pallas as pl
from jax.experimental.pallas import tpu as pltpu
from jax.experimental.pallas import tpu_sc as plsc

N = 10000
E = 320000
D_IN = 128
H = 64

NC = 2
NS = 16
NW = NC * NS
L = 16

CB = 128
C_TOT = 2560
E_PAD = C_TOT * CB
NCU = 2
Q0 = C_TOT // NW
N_ACC = 10240
CH_PER_SUB = N_ACC // CB // NS
N_SUB = N_ACC // NS
NBUF = 4
PF = 2

BLK = 2048
NBLK = N_ACC // BLK


@functools.lru_cache(maxsize=None)
def _make_sc_agg(D, with_count, two_tables=False):
    mesh = plsc.VectorSubcoreMesh(core_axis_name="c", subcore_axis_name="s",
                                  num_cores=NCU)
    QMAX = C_TOT // NS if two_tables else Q0
    outs = [jax.ShapeDtypeStruct((NCU, N_ACC, D), jnp.float32)]
    if with_count:
        outs.append(jax.ShapeDtypeStruct((NCU, N_ACC), jnp.float32))
    scratch = [
        pltpu.VMEM((QMAX, CB), jnp.int32),
        pltpu.VMEM((QMAX, CB), jnp.int32),
    ]
    scratch += [pltpu.VMEM((CB, D), jnp.float32) for _ in range(NBUF)]
    scratch.append(pltpu.VMEM_SHARED((N_ACC, D), jnp.float32))
    if with_count:
        scratch.append(pltpu.VMEM((CB,), jnp.float32))
        scratch.append(pltpu.VMEM((N_SUB,), jnp.float32))
        scratch.append(pltpu.VMEM_SHARED((N_ACC,), jnp.float32))
    scratch += [pltpu.SemaphoreType.DMA for _ in range(2 * NBUF)]
    if with_count:
        scratch.append(pltpu.SemaphoreType.DMA)

    @functools.partial(
        pl.kernel,
        mesh=mesh,
        out_type=tuple(outs) if with_count else outs[0],
        scratch_types=scratch,
        compiler_params=pltpu.CompilerParams(use_tc_tiling_on_sc=False),
    )
    def agg(*args):
        tab, srcs, dsts, *rest = args
        if with_count:
            (out, cnt_out, src_v, dst_v, *bufs, acc, ones_v, zc_v, cacc) = (
                rest[:-(2 * NBUF + 1)])
            sems = rest[-(2 * NBUF + 1):]
            gs, ss, csem = sems[:NBUF], sems[NBUF:2 * NBUF], sems[-1]
        else:
            (out, src_v, dst_v, *bufs, acc) = rest[:-(2 * NBUF)]
            sems = rest[-(2 * NBUF):]
            gs, ss = sems[:NBUF], sems[NBUF:]
            cnt_out = ones_v = zc_v = cacc = csem = None

        c = lax.axis_index("c")
        s = lax.axis_index("s")

        with jax.named_scope("agg_zero"):
            @pl.loop(0, CB)
            def _(i):
                @pl.loop(0, D // L)
                def _(k):
                    bufs[0][i, pl.ds(k * L, L)] = jnp.zeros((L,), jnp.float32)

            if with_count:
                @pl.loop(0, CB // L)
                def _(k):
                    ones_v[pl.ds(k * L, L)] = jnp.full((L,), 1.0, jnp.float32)

                @pl.loop(0, N_SUB // L)
                def _(k):
                    zc_v[pl.ds(k * L, L)] = jnp.zeros((L,), jnp.float32)

            for t in range(CH_PER_SUB):
                ch = s * CH_PER_SUB + t
                pltpu.sync_copy(bufs[0], acc.at[pl.ds(ch * CB, CB)])
            if with_count:
                pltpu.sync_copy(zc_v, cacc.at[pl.ds(s * N_SUB, N_SUB)])

        with jax.named_scope("agg_barrier1"):
            plsc.subcore_barrier()

        def run(nch, base, do_count):
            with jax.named_scope("agg_idx"):
                pltpu.sync_copy(srcs.at[pl.ds(base, nch)],
                                src_v.at[pl.ds(0, nch)])
                pltpu.sync_copy(dsts.at[pl.ds(base, nch)],
                                dst_v.at[pl.ds(0, nch)])
                if two_tables:
                    @pl.loop(0, nch)
                    def _(j):
                        @pl.loop(0, CB // L)
                        def _(k):
                            v = src_v[j, pl.ds(k * L, L)]
                            src_v[j, pl.ds(k * L, L)] = v + v + c

            for k in range(PF):
                pltpu.async_copy(tab.at[src_v.at[k]], bufs[k], gs[k])

            def ring_body(g):
                for b in range(NBUF):
                    j = g + b
                    b2 = (b + PF) % NBUF

                    @pl.when(j >= NBUF - PF)
                    def _():
                        pltpu.make_async_copy(
                            bufs[b2], acc.at[dst_v.at[j - (NBUF - PF)]],
                            ss[b2]).wait()

                    @pl.when(j + PF < nch)
                    def _():
                        pltpu.async_copy(tab.at[src_v.at[j + PF]], bufs[b2],
                                         gs[b2])

                    pltpu.make_async_copy(tab.at[src_v.at[j]], bufs[b],
                                          gs[b]).wait()
                    pltpu.async_copy(bufs[b], acc.at[dst_v.at[j]], ss[b],
                                     add=True)
                    if do_count:
                        pltpu.async_copy(ones_v, cacc.at[dst_v.at[j]], csem,
                                         add=True)

            with jax.named_scope("agg_ring"):
                @pl.loop(0, nch, step=NBUF)
                def _(g):
                    ring_body(g)

            with jax.named_scope("agg_drain"):
                for k in range(nch - NBUF + PF, nch):
                    pltpu.make_async_copy(bufs[k % NBUF],
                                          acc.at[dst_v.at[k]],
                                          ss[k % NBUF]).wait()
                if do_count:
                    @pl.loop(0, nch)
                    def _(j):
                        pltpu.make_async_copy(ones_v, cacc.at[dst_v.at[j]],
                                              csem).wait()

        if two_tables:
            if with_count:
                @pl.when(c == 0)
                def _():
                    run(QMAX, s * QMAX, True)

                @pl.when(c == 1)
                def _():
                    run(QMAX, s * QMAX, False)
            else:
                run(QMAX, s * QMAX, False)
        else:
            run(Q0, (c * NS + s) * Q0, with_count)

        with jax.named_scope("agg_barrier2"):
            plsc.subcore_barrier()

        with jax.named_scope("agg_dump"):
            for t in range(CH_PER_SUB):
                ch = s * CH_PER_SUB + t
                pltpu.sync_copy(acc.at[pl.ds(ch * CB, CB)],
                                out.at[c, pl.ds(ch * CB, CB)])
            if with_count:
                pltpu.sync_copy(cacc.at[pl.ds(s * N_SUB, N_SUB)],
                                cnt_out.at[c, pl.ds(s * N_SUB, N_SUB)])

    return agg


def _tc_a_body(x_ref, w_ref, b_ref, o_ref):
    o_ref[...] = (jnp.dot(x_ref[...], w_ref[...],
                          preferred_element_type=jnp.float32) + b_ref[...])


def _tc_a(x, W1_l, b1):
    return pl.pallas_call(
        _tc_a_body,
        grid=(NBLK,),
        in_specs=[
            pl.BlockSpec((BLK, D_IN), lambda i: (i, 0)),
            pl.BlockSpec((D_IN, 2 * H), lambda i: (0, 0)),
            pl.BlockSpec((1, 2 * H), lambda i: (0, 0)),
        ],
        out_specs=pl.BlockSpec((BLK, 2 * H), lambda i: (i, 0)),
        out_shape=jax.ShapeDtypeStruct((N, 2 * H), jnp.float32),
    )(x, W1_l, b1.reshape(1, 2 * H))


def _rcnt(c_ref):
    cnt = jnp.maximum(c_ref[0] + c_ref[1], 1.0)
    return (1.0 / cnt)[:, None]


def _tc_b_body(p_ref, a_ref, c_ref, w1ra_ref, w1rb_ref, w2_ref,
               b2_ref, o1_ref, o2_ref):
    rc = _rcnt(c_ref)
    m1a = a_ref[0] * rc
    m1b = a_ref[1] * rc
    h1 = jnp.maximum(
        p_ref[...]
        + jnp.dot(m1a, w1ra_ref[...], preferred_element_type=jnp.float32)
        + jnp.dot(m1b, w1rb_ref[...], preferred_element_type=jnp.float32),
        0.0)
    y2 = (jnp.dot(h1, w2_ref[...], preferred_element_type=jnp.float32)
          + b2_ref[...])
    o1_ref[...] = y2[:, :H]
    o2_ref[...] = y2[:, H:]


def _tc_b(p_l1, agg1, cntp, W1r_a, W1r_b, W2cat, b2cat):
    return pl.pallas_call(
        _tc_b_body,
        grid=(NBLK,),
        in_specs=[
            pl.BlockSpec((BLK, 2 * H), lambda i: (i, 0)),
            pl.BlockSpec((NCU, BLK, H), lambda i: (0, i, 0)),
            pl.BlockSpec((NCU, BLK), lambda i: (0, i)),
            pl.BlockSpec((H, 2 * H), lambda i: (0, 0)),
            pl.BlockSpec((H, 2 * H), lambda i: (0, 0)),
            pl.BlockSpec((2 * H, 2 * H), lambda i: (0, 0)),
            pl.BlockSpec((1, 2 * H), lambda i: (0, 0)),
        ],
        out_specs=[
            pl.BlockSpec((BLK, H), lambda i: (i, 0)),
            pl.BlockSpec((BLK, H), lambda i: (i, 0)),
        ],
        out_shape=[
            jax.ShapeDtypeStruct((N, H), jnp.float32),
            jax.ShapeDtypeStruct((N, H), jnp.float32),
        ],
    )(p_l1, agg1, cntp, W1r_a, W1r_b, W2cat, b2cat)


def _tc_c_body(p_ref, a_ref, c_ref, w_ref, b_ref, oh_ref, op_ref):
    m2 = (a_ref[0] + a_ref[1]) * _rcnt(c_ref)
    h2 = jnp.maximum(p_ref[...] + m2, 0.0)
    oh_ref[...] = h2
    op_ref[...] = (jnp.dot(h2, w_ref[...],
                           preferred_element_type=jnp.float32) + b_ref[...])


def _tc_c(p_l2b, agg2, cntp, Wmulv_l, bmulv):
    return pl.pallas_call(
        _tc_c_body,
        grid=(NBLK,),
        in_specs=[
            pl.BlockSpec((BLK, H), lambda i: (i, 0)),
            pl.BlockSpec((NCU, BLK, H), lambda i: (0, i, 0)),
            pl.BlockSpec((NCU, BLK), lambda i: (0, i)),
            pl.BlockSpec((H, 2 * H), lambda i: (0, 0)),
            pl.BlockSpec((1, 2 * H), lambda i: (0, 0)),
        ],
        out_specs=[
            pl.BlockSpec((BLK, H), lambda i: (i, 0)),
            pl.BlockSpec((BLK, 2 * H), lambda i: (i, 0)),
        ],
        out_shape=[
            jax.ShapeDtypeStruct((N, H), jnp.float32),
            jax.ShapeDtypeStruct((N, 2 * H), jnp.float32),
        ],
    )(p_l2b, agg2, cntp, Wmulv_l, bmulv)


def _tc_d_body(p_ref, a_ref, c_ref, w_ref, omu_ref, olv_ref):
    m3 = (a_ref[0] + a_ref[1]) * _rcnt(c_ref)
    res = p_ref[...] + jnp.dot(m3, w_ref[...],
                               preferred_element_type=jnp.float32)
    omu_ref[...] = res[:, :H]
    olv_ref[...] = res[:, H:]


def _tc_d(p_mulv, agg3, cntp, Wmulv_r):
    return pl.pallas_call(
        _tc_d_body,
        grid=(NBLK,),
        in_specs=[
            pl.BlockSpec((BLK, 2 * H), lambda i: (i, 0)),
            pl.BlockSpec((NCU, BLK, H), lambda i: (0, i, 0)),
            pl.BlockSpec((NCU, BLK), lambda i: (0, i)),
            pl.BlockSpec((H, 2 * H), lambda i: (0, 0)),
        ],
        out_specs=[
            pl.BlockSpec((BLK, H), lambda i: (i, 0)),
            pl.BlockSpec((BLK, H), lambda i: (i, 0)),
        ],
        out_shape=[
            jax.ShapeDtypeStruct((N, H), jnp.float32),
            jax.ShapeDtypeStruct((N, H), jnp.float32),
        ],
    )(p_mulv, agg3, cntp, Wmulv_r)


def kernel(x, edge_index, W1_l, b1, W1_r, W2_l, b2, W2_r,
           Wmu_l, bmu, Wmu_r, Wlv_l, blv, Wlv_r):
    src = edge_index[0].astype(jnp.int32)
    dst = edge_index[1].astype(jnp.int32)
    pad = E_PAD - E
    it = jnp.arange(pad, dtype=jnp.int32)
    srcs = jnp.concatenate([src, it % N]).reshape(C_TOT, CB)
    dsts = jnp.concatenate([dst, N + it % (N_ACC - N)]).reshape(C_TOT, CB)

    W2cat = jnp.concatenate([W2_l, W2_r], axis=1)
    b2cat = jnp.concatenate([b2, jnp.zeros((H,), jnp.float32)]).reshape(1, 2 * H)
    Wmulv_l = jnp.concatenate([Wmu_l, Wlv_l], axis=1)
    bmulv = jnp.concatenate([bmu, blv]).reshape(1, 2 * H)
    Wmulv_r = jnp.concatenate([Wmu_r, Wlv_r], axis=1)

    agg1, cntp = _make_sc_agg(H, True, True)(x.reshape(2 * N, H), srcs, dsts)
    p_l1 = _tc_a(x, W1_l, b1)
    p_l2b, t2 = _tc_b(p_l1, agg1, cntp, W1_r[:H], W1_r[H:], W2cat, b2cat)
    agg2 = _make_sc_agg(H, False)(t2, srcs, dsts)
    h2, p_mulv = _tc_c(p_l2b, agg2, cntp, Wmulv_l, bmulv)
    agg3 = _make_sc_agg(H, False)(h2, srcs, dsts)
    mu, lv = _tc_d(p_mulv, agg3, cntp, Wmulv_r)
    return (mu, lv)


# speedup vs baseline: 3.7418x; 1.0129x over previous
import functools

import jax
import jax.numpy as jnp
from jax import lax
from jax.experimental import pallas as pl
from jax.experimental.pallas import tpu as pltpu
from jax.experimental.pallas import tpu_sc as plsc

N = 10000
E = 320000
D_IN = 128
H = 64

NC = 2
NS = 16
NW = NC * NS
L = 16

CB = 128
C_TOT = 2560
E_PAD = C_TOT * CB
NCU = 2
Q0 = C_TOT // NW
N_ACC = 10240
CH_PER_SUB = N_ACC // CB // NS
N_SUB = N_ACC // NS
NBUF = 5
PF = 2

BLK = 2048
NBLK = N_ACC // BLK


@functools.lru_cache(maxsize=None)
def _make_sc_agg(D, with_count, two_tables=False):
    mesh = plsc.VectorSubcoreMesh(core_axis_name="c", subcore_axis_name="s",
                                  num_cores=NCU)
    QMAX = C_TOT // NS if two_tables else Q0
    outs = [jax.ShapeDtypeStruct((NCU, N_ACC, D), jnp.float32)]
    if with_count:
        outs.append(jax.ShapeDtypeStruct((NCU, N_ACC), jnp.float32))
    scratch = [
        pltpu.VMEM((QMAX, CB), jnp.int32),
        pltpu.VMEM((QMAX, CB), jnp.int32),
    ]
    scratch += [pltpu.VMEM((CB, D), jnp.float32) for _ in range(NBUF)]
    scratch.append(pltpu.VMEM_SHARED((N_ACC, D), jnp.float32))
    if with_count:
        scratch.append(pltpu.VMEM((CB,), jnp.float32))
        scratch.append(pltpu.VMEM((N_SUB,), jnp.float32))
        scratch.append(pltpu.VMEM_SHARED((N_ACC,), jnp.float32))
    scratch += [pltpu.SemaphoreType.DMA for _ in range(2 * NBUF)]
    if with_count:
        scratch.append(pltpu.SemaphoreType.DMA)

    @functools.partial(
        pl.kernel,
        mesh=mesh,
        out_type=tuple(outs) if with_count else outs[0],
        scratch_types=scratch,
        compiler_params=pltpu.CompilerParams(use_tc_tiling_on_sc=False),
    )
    def agg(*args):
        tab, srcs, dsts, *rest = args
        if with_count:
            (out, cnt_out, src_v, dst_v, *bufs, acc, ones_v, zc_v, cacc) = (
                rest[:-(2 * NBUF + 1)])
            sems = rest[-(2 * NBUF + 1):]
            gs, ss, csem = sems[:NBUF], sems[NBUF:2 * NBUF], sems[-1]
        else:
            (out, src_v, dst_v, *bufs, acc) = rest[:-(2 * NBUF)]
            sems = rest[-(2 * NBUF):]
            gs, ss = sems[:NBUF], sems[NBUF:]
            cnt_out = ones_v = zc_v = cacc = csem = None

        c = lax.axis_index("c")
        s = lax.axis_index("s")

        with jax.named_scope("agg_zero"):
            @pl.loop(0, CB)
            def _(i):
                @pl.loop(0, D // L)
                def _(k):
                    bufs[0][i, pl.ds(k * L, L)] = jnp.zeros((L,), jnp.float32)

            if with_count:
                @pl.loop(0, CB // L)
                def _(k):
                    ones_v[pl.ds(k * L, L)] = jnp.full((L,), 1.0, jnp.float32)

                @pl.loop(0, N_SUB // L)
                def _(k):
                    zc_v[pl.ds(k * L, L)] = jnp.zeros((L,), jnp.float32)

            for t in range(CH_PER_SUB):
                ch = s * CH_PER_SUB + t
                pltpu.sync_copy(bufs[0], acc.at[pl.ds(ch * CB, CB)])
            if with_count:
                pltpu.sync_copy(zc_v, cacc.at[pl.ds(s * N_SUB, N_SUB)])

        with jax.named_scope("agg_barrier1"):
            plsc.subcore_barrier()

        def run(nch, base, do_count):
            with jax.named_scope("agg_idx"):
                pltpu.sync_copy(srcs.at[pl.ds(base, nch)],
                                src_v.at[pl.ds(0, nch)])
                pltpu.sync_copy(dsts.at[pl.ds(base, nch)],
                                dst_v.at[pl.ds(0, nch)])
                if two_tables:
                    @pl.loop(0, nch)
                    def _(j):
                        @pl.loop(0, CB // L)
                        def _(k):
                            v = src_v[j, pl.ds(k * L, L)]
                            src_v[j, pl.ds(k * L, L)] = v + v + c

            for k in range(PF):
                pltpu.async_copy(tab.at[src_v.at[k]], bufs[k], gs[k])

            def ring_body(g):
                for b in range(NBUF):
                    j = g + b
                    b2 = (b + PF) % NBUF

                    @pl.when(j >= NBUF - PF)
                    def _():
                        pltpu.make_async_copy(
                            bufs[b2], acc.at[dst_v.at[j - (NBUF - PF)]],
                            ss[b2]).wait()

                    @pl.when(j + PF < nch)
                    def _():
                        pltpu.async_copy(tab.at[src_v.at[j + PF]], bufs[b2],
                                         gs[b2])

                    pltpu.make_async_copy(tab.at[src_v.at[j]], bufs[b],
                                          gs[b]).wait()
                    pltpu.async_copy(bufs[b], acc.at[dst_v.at[j]], ss[b],
                                     add=True)
                    if do_count:
                        pltpu.async_copy(ones_v, cacc.at[dst_v.at[j]], csem,
                                         add=True)

            with jax.named_scope("agg_ring"):
                @pl.loop(0, nch, step=NBUF)
                def _(g):
                    ring_body(g)

            with jax.named_scope("agg_drain"):
                for k in range(nch - NBUF + PF, nch):
                    pltpu.make_async_copy(bufs[k % NBUF],
                                          acc.at[dst_v.at[k]],
                                          ss[k % NBUF]).wait()
                if do_count:
                    @pl.loop(0, nch)
                    def _(j):
                        pltpu.make_async_copy(ones_v, cacc.at[dst_v.at[j]],
                                              csem).wait()

        if two_tables:
            if with_count:
                @pl.when(c == 0)
                def _():
                    run(QMAX, s * QMAX, True)

                @pl.when(c == 1)
                def _():
                    run(QMAX, s * QMAX, False)
            else:
                run(QMAX, s * QMAX, False)
        else:
            run(Q0, (c * NS + s) * Q0, with_count)

        with jax.named_scope("agg_barrier2"):
            plsc.subcore_barrier()

        with jax.named_scope("agg_dump"):
            for t in range(CH_PER_SUB):
                ch = s * CH_PER_SUB + t
                pltpu.sync_copy(acc.at[pl.ds(ch * CB, CB)],
                                out.at[c, pl.ds(ch * CB, CB)])
            if with_count:
                pltpu.sync_copy(cacc.at[pl.ds(s * N_SUB, N_SUB)],
                                cnt_out.at[c, pl.ds(s * N_SUB, N_SUB)])

    return agg


def _tc_a_body(x_ref, w_ref, b_ref, o_ref):
    o_ref[...] = (jnp.dot(x_ref[...], w_ref[...],
                          preferred_element_type=jnp.float32) + b_ref[...])


def _tc_a(x, W1_l, b1):
    return pl.pallas_call(
        _tc_a_body,
        grid=(NBLK,),
        in_specs=[
            pl.BlockSpec((BLK, D_IN), lambda i: (i, 0)),
            pl.BlockSpec((D_IN, 2 * H), lambda i: (0, 0)),
            pl.BlockSpec((1, 2 * H), lambda i: (0, 0)),
        ],
        out_specs=pl.BlockSpec((BLK, 2 * H), lambda i: (i, 0)),
        out_shape=jax.ShapeDtypeStruct((N, 2 * H), jnp.float32),
    )(x, W1_l, b1.reshape(1, 2 * H))


def _rcnt(c_ref):
    cnt = jnp.maximum(c_ref[0] + c_ref[1], 1.0)
    return (1.0 / cnt)[:, None]


def _tc_b_body(p_ref, a_ref, c_ref, w1ra_ref, w1rb_ref, w2_ref,
               b2_ref, o1_ref, o2_ref):
    rc = _rcnt(c_ref)
    m1a = a_ref[0] * rc
    m1b = a_ref[1] * rc
    h1 = jnp.maximum(
        p_ref[...]
        + jnp.dot(m1a, w1ra_ref[...], preferred_element_type=jnp.float32)
        + jnp.dot(m1b, w1rb_ref[...], preferred_element_type=jnp.float32),
        0.0)
    y2 = (jnp.dot(h1, w2_ref[...], preferred_element_type=jnp.float32)
          + b2_ref[...])
    o1_ref[...] = y2[:, :H]
    o2_ref[...] = y2[:, H:]


def _tc_b(p_l1, agg1, cntp, W1r_a, W1r_b, W2cat, b2cat):
    return pl.pallas_call(
        _tc_b_body,
        grid=(NBLK,),
        in_specs=[
            pl.BlockSpec((BLK, 2 * H), lambda i: (i, 0)),
            pl.BlockSpec((NCU, BLK, H), lambda i: (0, i, 0)),
            pl.BlockSpec((NCU, BLK), lambda i: (0, i)),
            pl.BlockSpec((H, 2 * H), lambda i: (0, 0)),
            pl.BlockSpec((H, 2 * H), lambda i: (0, 0)),
            pl.BlockSpec((2 * H, 2 * H), lambda i: (0, 0)),
            pl.BlockSpec((1, 2 * H), lambda i: (0, 0)),
        ],
        out_specs=[
            pl.BlockSpec((BLK, H), lambda i: (i, 0)),
            pl.BlockSpec((BLK, H), lambda i: (i, 0)),
        ],
        out_shape=[
            jax.ShapeDtypeStruct((N, H), jnp.float32),
            jax.ShapeDtypeStruct((N, H), jnp.float32),
        ],
    )(p_l1, agg1, cntp, W1r_a, W1r_b, W2cat, b2cat)


def _tc_c_body(p_ref, a_ref, c_ref, w_ref, b_ref, oh_ref, op_ref):
    m2 = (a_ref[0] + a_ref[1]) * _rcnt(c_ref)
    h2 = jnp.maximum(p_ref[...] + m2, 0.0)
    oh_ref[...] = h2
    op_ref[...] = (jnp.dot(h2, w_ref[...],
                           preferred_element_type=jnp.float32) + b_ref[...])


def _tc_c(p_l2b, agg2, cntp, Wmulv_l, bmulv):
    return pl.pallas_call(
        _tc_c_body,
        grid=(NBLK,),
        in_specs=[
            pl.BlockSpec((BLK, H), lambda i: (i, 0)),
            pl.BlockSpec((NCU, BLK, H), lambda i: (0, i, 0)),
            pl.BlockSpec((NCU, BLK), lambda i: (0, i)),
            pl.BlockSpec((H, 2 * H), lambda i: (0, 0)),
            pl.BlockSpec((1, 2 * H), lambda i: (0, 0)),
        ],
        out_specs=[
            pl.BlockSpec((BLK, H), lambda i: (i, 0)),
            pl.BlockSpec((BLK, 2 * H), lambda i: (i, 0)),
        ],
        out_shape=[
            jax.ShapeDtypeStruct((N, H), jnp.float32),
            jax.ShapeDtypeStruct((N, 2 * H), jnp.float32),
        ],
    )(p_l2b, agg2, cntp, Wmulv_l, bmulv)


def _tc_d_body(p_ref, a_ref, c_ref, w_ref, omu_ref, olv_ref):
    m3 = (a_ref[0] + a_ref[1]) * _rcnt(c_ref)
    res = p_ref[...] + jnp.dot(m3, w_ref[...],
                               preferred_element_type=jnp.float32)
    omu_ref[...] = res[:, :H]
    olv_ref[...] = res[:, H:]


def _tc_d(p_mulv, agg3, cntp, Wmulv_r):
    return pl.pallas_call(
        _tc_d_body,
        grid=(NBLK,),
        in_specs=[
            pl.BlockSpec((BLK, 2 * H), lambda i: (i, 0)),
            pl.BlockSpec((NCU, BLK, H), lambda i: (0, i, 0)),
            pl.BlockSpec((NCU, BLK), lambda i: (0, i)),
            pl.BlockSpec((H, 2 * H), lambda i: (0, 0)),
        ],
        out_specs=[
            pl.BlockSpec((BLK, H), lambda i: (i, 0)),
            pl.BlockSpec((BLK, H), lambda i: (i, 0)),
        ],
        out_shape=[
            jax.ShapeDtypeStruct((N, H), jnp.float32),
            jax.ShapeDtypeStruct((N, H), jnp.float32),
        ],
    )(p_mulv, agg3, cntp, Wmulv_r)


def kernel(x, edge_index, W1_l, b1, W1_r, W2_l, b2, W2_r,
           Wmu_l, bmu, Wmu_r, Wlv_l, blv, Wlv_r):
    src = edge_index[0].astype(jnp.int32)
    dst = edge_index[1].astype(jnp.int32)
    pad = E_PAD - E
    it = jnp.arange(pad, dtype=jnp.int32)
    srcs = jnp.concatenate([src, it % N]).reshape(C_TOT, CB)
    dsts = jnp.concatenate([dst, N + it % (N_ACC - N)]).reshape(C_TOT, CB)

    W2cat = jnp.concatenate([W2_l, W2_r], axis=1)
    b2cat = jnp.concatenate([b2, jnp.zeros((H,), jnp.float32)]).reshape(1, 2 * H)
    Wmulv_l = jnp.concatenate([Wmu_l, Wlv_l], axis=1)
    bmulv = jnp.concatenate([bmu, blv]).reshape(1, 2 * H)
    Wmulv_r = jnp.concatenate([Wmu_r, Wlv_r], axis=1)

    agg1, cntp = _make_sc_agg(H, True, True)(x.reshape(2 * N, H), srcs, dsts)
    p_l1 = _tc_a(x, W1_l, b1)
    p_l2b, t2 = _tc_b(p_l1, agg1, cntp, W1_r[:H], W1_r[H:], W2cat, b2cat)
    agg2 = _make_sc_agg(H, False)(t2, srcs, dsts)
    h2, p_mulv = _tc_c(p_l2b, agg2, cntp, Wmulv_l, bmulv)
    agg3 = _make_sc_agg(H, False)(h2, srcs, dsts)
    mu, lv = _tc_d(p_mulv, agg3, cntp, Wmulv_r)
    return (mu, lv)


# speedup vs baseline: 3.8306x; 1.0237x over previous
import functools

import jax
import jax.numpy as jnp
from jax import lax
from jax.experimental import pallas as pl
from jax.experimental.pallas import tpu as pltpu
from jax.experimental.pallas import tpu_sc as plsc

N = 10000
E = 320000
D_IN = 128
H = 64

NC = 2
NS = 16
NW = NC * NS
L = 16

CB = 128
C_TOT = 2560
E_PAD = C_TOT * CB
NCU = 2
Q0 = C_TOT // NW
N_ACC = 10240
CH_PER_SUB = N_ACC // CB // NS
N_SUB = N_ACC // NS
NBUF = 5
PF = 3

BLK = 2048
NBLK = N_ACC // BLK


@functools.lru_cache(maxsize=None)
def _make_sc_agg(D, with_count, two_tables=False):
    mesh = plsc.VectorSubcoreMesh(core_axis_name="c", subcore_axis_name="s",
                                  num_cores=NCU)
    QMAX = C_TOT // NS if two_tables else Q0
    outs = [jax.ShapeDtypeStruct((NCU, N_ACC, D), jnp.float32)]
    if with_count:
        outs.append(jax.ShapeDtypeStruct((NCU, N_ACC), jnp.float32))
    scratch = [
        pltpu.VMEM((QMAX, CB), jnp.int32),
        pltpu.VMEM((QMAX, CB), jnp.int32),
    ]
    scratch += [pltpu.VMEM((CB, D), jnp.float32) for _ in range(NBUF)]
    scratch.append(pltpu.VMEM_SHARED((N_ACC, D), jnp.float32))
    if with_count:
        scratch.append(pltpu.VMEM((CB,), jnp.float32))
        scratch.append(pltpu.VMEM((N_SUB,), jnp.float32))
        scratch.append(pltpu.VMEM_SHARED((N_ACC,), jnp.float32))
    scratch += [pltpu.SemaphoreType.DMA for _ in range(2 * NBUF)]
    if with_count:
        scratch.append(pltpu.SemaphoreType.DMA)

    @functools.partial(
        pl.kernel,
        mesh=mesh,
        out_type=tuple(outs) if with_count else outs[0],
        scratch_types=scratch,
        compiler_params=pltpu.CompilerParams(use_tc_tiling_on_sc=False),
    )
    def agg(*args):
        tab, srcs, dsts, *rest = args
        if with_count:
            (out, cnt_out, src_v, dst_v, *bufs, acc, ones_v, zc_v, cacc) = (
                rest[:-(2 * NBUF + 1)])
            sems = rest[-(2 * NBUF + 1):]
            gs, ss, csem = sems[:NBUF], sems[NBUF:2 * NBUF], sems[-1]
        else:
            (out, src_v, dst_v, *bufs, acc) = rest[:-(2 * NBUF)]
            sems = rest[-(2 * NBUF):]
            gs, ss = sems[:NBUF], sems[NBUF:]
            cnt_out = ones_v = zc_v = cacc = csem = None

        c = lax.axis_index("c")
        s = lax.axis_index("s")

        with jax.named_scope("agg_zero"):
            @pl.loop(0, CB)
            def _(i):
                @pl.loop(0, D // L)
                def _(k):
                    bufs[0][i, pl.ds(k * L, L)] = jnp.zeros((L,), jnp.float32)

            if with_count:
                @pl.loop(0, CB // L)
                def _(k):
                    ones_v[pl.ds(k * L, L)] = jnp.full((L,), 1.0, jnp.float32)

                @pl.loop(0, N_SUB // L)
                def _(k):
                    zc_v[pl.ds(k * L, L)] = jnp.zeros((L,), jnp.float32)

            for t in range(CH_PER_SUB):
                ch = s * CH_PER_SUB + t
                pltpu.sync_copy(bufs[0], acc.at[pl.ds(ch * CB, CB)])
            if with_count:
                pltpu.sync_copy(zc_v, cacc.at[pl.ds(s * N_SUB, N_SUB)])

        with jax.named_scope("agg_barrier1"):
            plsc.subcore_barrier()

        def run(nch, base, do_count):
            with jax.named_scope("agg_idx"):
                pltpu.sync_copy(srcs.at[pl.ds(base, nch)],
                                src_v.at[pl.ds(0, nch)])
                pltpu.sync_copy(dsts.at[pl.ds(base, nch)],
                                dst_v.at[pl.ds(0, nch)])
                if two_tables:
                    @pl.loop(0, nch)
                    def _(j):
                        @pl.loop(0, CB // L)
                        def _(k):
                            v = src_v[j, pl.ds(k * L, L)]
                            src_v[j, pl.ds(k * L, L)] = v + v + c

            for k in range(PF):
                pltpu.async_copy(tab.at[src_v.at[k]], bufs[k], gs[k])

            def ring_body(g):
                for b in range(NBUF):
                    j = g + b
                    b2 = (b + PF) % NBUF

                    @pl.when(j >= NBUF - PF)
                    def _():
                        pltpu.make_async_copy(
                            bufs[b2], acc.at[dst_v.at[j - (NBUF - PF)]],
                            ss[b2]).wait()

                    @pl.when(j + PF < nch)
                    def _():
                        pltpu.async_copy(tab.at[src_v.at[j + PF]], bufs[b2],
                                         gs[b2])

                    pltpu.make_async_copy(tab.at[src_v.at[j]], bufs[b],
                                          gs[b]).wait()
                    pltpu.async_copy(bufs[b], acc.at[dst_v.at[j]], ss[b],
                                     add=True)
                    if do_count:
                        pltpu.async_copy(ones_v, cacc.at[dst_v.at[j]], csem,
                                         add=True)

            with jax.named_scope("agg_ring"):
                @pl.loop(0, nch, step=NBUF)
                def _(g):
                    ring_body(g)

            with jax.named_scope("agg_drain"):
                for k in range(nch - NBUF + PF, nch):
                    pltpu.make_async_copy(bufs[k % NBUF],
                                          acc.at[dst_v.at[k]],
                                          ss[k % NBUF]).wait()
                if do_count:
                    @pl.loop(0, nch)
                    def _(j):
                        pltpu.make_async_copy(ones_v, cacc.at[dst_v.at[j]],
                                              csem).wait()

        if two_tables:
            if with_count:
                @pl.when(c == 0)
                def _():
                    run(QMAX, s * QMAX, True)

                @pl.when(c == 1)
                def _():
                    run(QMAX, s * QMAX, False)
            else:
                run(QMAX, s * QMAX, False)
        else:
            run(Q0, (c * NS + s) * Q0, with_count)

        with jax.named_scope("agg_barrier2"):
            plsc.subcore_barrier()

        with jax.named_scope("agg_dump"):
            for t in range(CH_PER_SUB):
                ch = s * CH_PER_SUB + t
                pltpu.sync_copy(acc.at[pl.ds(ch * CB, CB)],
                                out.at[c, pl.ds(ch * CB, CB)])
            if with_count:
                pltpu.sync_copy(cacc.at[pl.ds(s * N_SUB, N_SUB)],
                                cnt_out.at[c, pl.ds(s * N_SUB, N_SUB)])

    return agg


def _tc_a_body(x_ref, w_ref, b_ref, o_ref):
    o_ref[...] = (jnp.dot(x_ref[...], w_ref[...],
                          preferred_element_type=jnp.float32) + b_ref[...])


def _tc_a(x, W1_l, b1):
    return pl.pallas_call(
        _tc_a_body,
        grid=(NBLK,),
        in_specs=[
            pl.BlockSpec((BLK, D_IN), lambda i: (i, 0)),
            pl.BlockSpec((D_IN, 2 * H), lambda i: (0, 0)),
            pl.BlockSpec((1, 2 * H), lambda i: (0, 0)),
        ],
        out_specs=pl.BlockSpec((BLK, 2 * H), lambda i: (i, 0)),
        out_shape=jax.ShapeDtypeStruct((N, 2 * H), jnp.float32),
    )(x, W1_l, b1.reshape(1, 2 * H))


def _rcnt(c_ref):
    cnt = jnp.maximum(c_ref[0] + c_ref[1], 1.0)
    return (1.0 / cnt)[:, None]


def _tc_b_body(p_ref, a_ref, c_ref, w1ra_ref, w1rb_ref, w2_ref,
               b2_ref, o1_ref, o2_ref):
    rc = _rcnt(c_ref)
    m1a = a_ref[0] * rc
    m1b = a_ref[1] * rc
    h1 = jnp.maximum(
        p_ref[...]
        + jnp.dot(m1a, w1ra_ref[...], preferred_element_type=jnp.float32)
        + jnp.dot(m1b, w1rb_ref[...], preferred_element_type=jnp.float32),
        0.0)
    y2 = (jnp.dot(h1, w2_ref[...], preferred_element_type=jnp.float32)
          + b2_ref[...])
    o1_ref[...] = y2[:, :H]
    o2_ref[...] = y2[:, H:]


def _tc_b(p_l1, agg1, cntp, W1r_a, W1r_b, W2cat, b2cat):
    return pl.pallas_call(
        _tc_b_body,
        grid=(NBLK,),
        in_specs=[
            pl.BlockSpec((BLK, 2 * H), lambda i: (i, 0)),
            pl.BlockSpec((NCU, BLK, H), lambda i: (0, i, 0)),
            pl.BlockSpec((NCU, BLK), lambda i: (0, i)),
            pl.BlockSpec((H, 2 * H), lambda i: (0, 0)),
            pl.BlockSpec((H, 2 * H), lambda i: (0, 0)),
            pl.BlockSpec((2 * H, 2 * H), lambda i: (0, 0)),
            pl.BlockSpec((1, 2 * H), lambda i: (0, 0)),
        ],
        out_specs=[
            pl.BlockSpec((BLK, H), lambda i: (i, 0)),
            pl.BlockSpec((BLK, H), lambda i: (i, 0)),
        ],
        out_shape=[
            jax.ShapeDtypeStruct((N, H), jnp.float32),
            jax.ShapeDtypeStruct((N, H), jnp.float32),
        ],
    )(p_l1, agg1, cntp, W1r_a, W1r_b, W2cat, b2cat)


def _tc_c_body(p_ref, a_ref, c_ref, w_ref, b_ref, oh_ref, op_ref):
    m2 = (a_ref[0] + a_ref[1]) * _rcnt(c_ref)
    h2 = jnp.maximum(p_ref[...] + m2, 0.0)
    oh_ref[...] = h2
    op_ref[...] = (jnp.dot(h2, w_ref[...],
                           preferred_element_type=jnp.float32) + b_ref[...])


def _tc_c(p_l2b, agg2, cntp, Wmulv_l, bmulv):
    return pl.pallas_call(
        _tc_c_body,
        grid=(NBLK,),
        in_specs=[
            pl.BlockSpec((BLK, H), lambda i: (i, 0)),
            pl.BlockSpec((NCU, BLK, H), lambda i: (0, i, 0)),
            pl.BlockSpec((NCU, BLK), lambda i: (0, i)),
            pl.BlockSpec((H, 2 * H), lambda i: (0, 0)),
            pl.BlockSpec((1, 2 * H), lambda i: (0, 0)),
        ],
        out_specs=[
            pl.BlockSpec((BLK, H), lambda i: (i, 0)),
            pl.BlockSpec((BLK, 2 * H), lambda i: (i, 0)),
        ],
        out_shape=[
            jax.ShapeDtypeStruct((N, H), jnp.float32),
            jax.ShapeDtypeStruct((N, 2 * H), jnp.float32),
        ],
    )(p_l2b, agg2, cntp, Wmulv_l, bmulv)


def _tc_d_body(p_ref, a_ref, c_ref, w_ref, omu_ref, olv_ref):
    m3 = (a_ref[0] + a_ref[1]) * _rcnt(c_ref)
    res = p_ref[...] + jnp.dot(m3, w_ref[...],
                               preferred_element_type=jnp.float32)
    omu_ref[...] = res[:, :H]
    olv_ref[...] = res[:, H:]


def _tc_d(p_mulv, agg3, cntp, Wmulv_r):
    return pl.pallas_call(
        _tc_d_body,
        grid=(NBLK,),
        in_specs=[
            pl.BlockSpec((BLK, 2 * H), lambda i: (i, 0)),
            pl.BlockSpec((NCU, BLK, H), lambda i: (0, i, 0)),
            pl.BlockSpec((NCU, BLK), lambda i: (0, i)),
            pl.BlockSpec((H, 2 * H), lambda i: (0, 0)),
        ],
        out_specs=[
            pl.BlockSpec((BLK, H), lambda i: (i, 0)),
            pl.BlockSpec((BLK, H), lambda i: (i, 0)),
        ],
        out_shape=[
            jax.ShapeDtypeStruct((N, H), jnp.float32),
            jax.ShapeDtypeStruct((N, H), jnp.float32),
        ],
    )(p_mulv, agg3, cntp, Wmulv_r)


def kernel(x, edge_index, W1_l, b1, W1_r, W2_l, b2, W2_r,
           Wmu_l, bmu, Wmu_r, Wlv_l, blv, Wlv_r):
    src = edge_index[0].astype(jnp.int32)
    dst = edge_index[1].astype(jnp.int32)
    pad = E_PAD - E
    it = jnp.arange(pad, dtype=jnp.int32)
    srcs = jnp.concatenate([src, it % N]).reshape(C_TOT, CB)
    dsts = jnp.concatenate([dst, N + it % (N_ACC - N)]).reshape(C_TOT, CB)

    W2cat = jnp.concatenate([W2_l, W2_r], axis=1)
    b2cat = jnp.concatenate([b2, jnp.zeros((H,), jnp.float32)]).reshape(1, 2 * H)
    Wmulv_l = jnp.concatenate([Wmu_l, Wlv_l], axis=1)
    bmulv = jnp.concatenate([bmu, blv]).reshape(1, 2 * H)
    Wmulv_r = jnp.concatenate([Wmu_r, Wlv_r], axis=1)

    agg1, cntp = _make_sc_agg(H, True, True)(x.reshape(2 * N, H), srcs, dsts)
    p_l1 = _tc_a(x, W1_l, b1)
    p_l2b, t2 = _tc_b(p_l1, agg1, cntp, W1_r[:H], W1_r[H:], W2cat, b2cat)
    agg2 = _make_sc_agg(H, False)(t2, srcs, dsts)
    h2, p_mulv = _tc_c(p_l2b, agg2, cntp, Wmulv_l, bmulv)
    agg3 = _make_sc_agg(H, False)(h2, srcs, dsts)
    mu, lv = _tc_d(p_mulv, agg3, cntp, Wmulv_r)
    return (mu, lv)


# speedup vs baseline: 3.8617x; 1.0081x over previous
import functools

import jax
import jax.numpy as jnp
from jax import lax
from jax.experimental import pallas as pl
from jax.experimental.pallas import tpu as pltpu
from jax.experimental.pallas import tpu_sc as plsc

N = 10000
E = 320000
D_IN = 128
H = 64

NC = 2
NS = 16
NW = NC * NS
L = 16

CB = 128
C_TOT = 2560
E_PAD = C_TOT * CB
NCU = 2
Q0 = C_TOT // NW
N_ACC = 10240
CH_PER_SUB = N_ACC // CB // NS
N_SUB = N_ACC // NS
NBUF = 5
PF = 4

BLK = 2048
NBLK = N_ACC // BLK


@functools.lru_cache(maxsize=None)
def _make_sc_agg(D, with_count, two_tables=False):
    mesh = plsc.VectorSubcoreMesh(core_axis_name="c", subcore_axis_name="s",
                                  num_cores=NCU)
    QMAX = C_TOT // NS if two_tables else Q0
    outs = [jax.ShapeDtypeStruct((NCU, N_ACC, D), jnp.float32)]
    if with_count:
        outs.append(jax.ShapeDtypeStruct((NCU, N_ACC), jnp.float32))
    scratch = [
        pltpu.VMEM((QMAX, CB), jnp.int32),
        pltpu.VMEM((QMAX, CB), jnp.int32),
    ]
    scratch += [pltpu.VMEM((CB, D), jnp.float32) for _ in range(NBUF)]
    scratch.append(pltpu.VMEM_SHARED((N_ACC, D), jnp.float32))
    if with_count:
        scratch.append(pltpu.VMEM((CB,), jnp.float32))
        scratch.append(pltpu.VMEM((N_SUB,), jnp.float32))
        scratch.append(pltpu.VMEM_SHARED((N_ACC,), jnp.float32))
    scratch += [pltpu.SemaphoreType.DMA for _ in range(2 * NBUF)]
    if with_count:
        scratch.append(pltpu.SemaphoreType.DMA)

    @functools.partial(
        pl.kernel,
        mesh=mesh,
        out_type=tuple(outs) if with_count else outs[0],
        scratch_types=scratch,
        compiler_params=pltpu.CompilerParams(use_tc_tiling_on_sc=False),
    )
    def agg(*args):
        tab, srcs, dsts, *rest = args
        if with_count:
            (out, cnt_out, src_v, dst_v, *bufs, acc, ones_v, zc_v, cacc) = (
                rest[:-(2 * NBUF + 1)])
            sems = rest[-(2 * NBUF + 1):]
            gs, ss, csem = sems[:NBUF], sems[NBUF:2 * NBUF], sems[-1]
        else:
            (out, src_v, dst_v, *bufs, acc) = rest[:-(2 * NBUF)]
            sems = rest[-(2 * NBUF):]
            gs, ss = sems[:NBUF], sems[NBUF:]
            cnt_out = ones_v = zc_v = cacc = csem = None

        c = lax.axis_index("c")
        s = lax.axis_index("s")

        with jax.named_scope("agg_zero"):
            @pl.loop(0, CB)
            def _(i):
                @pl.loop(0, D // L)
                def _(k):
                    bufs[0][i, pl.ds(k * L, L)] = jnp.zeros((L,), jnp.float32)

            if with_count:
                @pl.loop(0, CB // L)
                def _(k):
                    ones_v[pl.ds(k * L, L)] = jnp.full((L,), 1.0, jnp.float32)

                @pl.loop(0, N_SUB // L)
                def _(k):
                    zc_v[pl.ds(k * L, L)] = jnp.zeros((L,), jnp.float32)

            for t in range(CH_PER_SUB):
                ch = s * CH_PER_SUB + t
                pltpu.sync_copy(bufs[0], acc.at[pl.ds(ch * CB, CB)])
            if with_count:
                pltpu.sync_copy(zc_v, cacc.at[pl.ds(s * N_SUB, N_SUB)])

        with jax.named_scope("agg_barrier1"):
            plsc.subcore_barrier()

        def run(nch, base, do_count):
            with jax.named_scope("agg_idx"):
                pltpu.sync_copy(srcs.at[pl.ds(base, nch)],
                                src_v.at[pl.ds(0, nch)])
                pltpu.sync_copy(dsts.at[pl.ds(base, nch)],
                                dst_v.at[pl.ds(0, nch)])
                if two_tables:
                    @pl.loop(0, nch)
                    def _(j):
                        @pl.loop(0, CB // L)
                        def _(k):
                            v = src_v[j, pl.ds(k * L, L)]
                            src_v[j, pl.ds(k * L, L)] = v + v + c

            for k in range(PF):
                pltpu.async_copy(tab.at[src_v.at[k]], bufs[k], gs[k])

            def ring_body(g):
                for b in range(NBUF):
                    j = g + b
                    b2 = (b + PF) % NBUF

                    @pl.when(j >= NBUF - PF)
                    def _():
                        pltpu.make_async_copy(
                            bufs[b2], acc.at[dst_v.at[j - (NBUF - PF)]],
                            ss[b2]).wait()

                    @pl.when(j + PF < nch)
                    def _():
                        pltpu.async_copy(tab.at[src_v.at[j + PF]], bufs[b2],
                                         gs[b2])

                    pltpu.make_async_copy(tab.at[src_v.at[j]], bufs[b],
                                          gs[b]).wait()
                    pltpu.async_copy(bufs[b], acc.at[dst_v.at[j]], ss[b],
                                     add=True)
                    if do_count:
                        pltpu.async_copy(ones_v, cacc.at[dst_v.at[j]], csem,
                                         add=True)

            with jax.named_scope("agg_ring"):
                @pl.loop(0, nch, step=NBUF)
                def _(g):
                    ring_body(g)

            with jax.named_scope("agg_drain"):
                for k in range(nch - NBUF + PF, nch):
                    pltpu.make_async_copy(bufs[k % NBUF],
                                          acc.at[dst_v.at[k]],
                                          ss[k % NBUF]).wait()
                if do_count:
                    @pl.loop(0, nch)
                    def _(j):
                        pltpu.make_async_copy(ones_v, cacc.at[dst_v.at[j]],
                                              csem).wait()

        if two_tables:
            if with_count:
                @pl.when(c == 0)
                def _():
                    run(QMAX, s * QMAX, True)

                @pl.when(c == 1)
                def _():
                    run(QMAX, s * QMAX, False)
            else:
                run(QMAX, s * QMAX, False)
        else:
            run(Q0, (c * NS + s) * Q0, with_count)

        with jax.named_scope("agg_barrier2"):
            plsc.subcore_barrier()

        with jax.named_scope("agg_dump"):
            for t in range(CH_PER_SUB):
                ch = s * CH_PER_SUB + t
                pltpu.sync_copy(acc.at[pl.ds(ch * CB, CB)],
                                out.at[c, pl.ds(ch * CB, CB)])
            if with_count:
                pltpu.sync_copy(cacc.at[pl.ds(s * N_SUB, N_SUB)],
                                cnt_out.at[c, pl.ds(s * N_SUB, N_SUB)])

    return agg


def _tc_a_body(x_ref, w_ref, b_ref, o_ref):
    o_ref[...] = (jnp.dot(x_ref[...], w_ref[...],
                          preferred_element_type=jnp.float32) + b_ref[...])


def _tc_a(x, W1_l, b1):
    return pl.pallas_call(
        _tc_a_body,
        grid=(NBLK,),
        in_specs=[
            pl.BlockSpec((BLK, D_IN), lambda i: (i, 0)),
            pl.BlockSpec((D_IN, 2 * H), lambda i: (0, 0)),
            pl.BlockSpec((1, 2 * H), lambda i: (0, 0)),
        ],
        out_specs=pl.BlockSpec((BLK, 2 * H), lambda i: (i, 0)),
        out_shape=jax.ShapeDtypeStruct((N, 2 * H), jnp.float32),
    )(x, W1_l, b1.reshape(1, 2 * H))


def _rcnt(c_ref):
    cnt = jnp.maximum(c_ref[0] + c_ref[1], 1.0)
    return (1.0 / cnt)[:, None]


def _tc_b_body(p_ref, a_ref, c_ref, w1ra_ref, w1rb_ref, w2_ref,
               b2_ref, o1_ref, o2_ref):
    rc = _rcnt(c_ref)
    m1a = a_ref[0] * rc
    m1b = a_ref[1] * rc
    h1 = jnp.maximum(
        p_ref[...]
        + jnp.dot(m1a, w1ra_ref[...], preferred_element_type=jnp.float32)
        + jnp.dot(m1b, w1rb_ref[...], preferred_element_type=jnp.float32),
        0.0)
    y2 = (jnp.dot(h1, w2_ref[...], preferred_element_type=jnp.float32)
          + b2_ref[...])
    o1_ref[...] = y2[:, :H]
    o2_ref[...] = y2[:, H:]


def _tc_b(p_l1, agg1, cntp, W1r_a, W1r_b, W2cat, b2cat):
    return pl.pallas_call(
        _tc_b_body,
        grid=(NBLK,),
        in_specs=[
            pl.BlockSpec((BLK, 2 * H), lambda i: (i, 0)),
            pl.BlockSpec((NCU, BLK, H), lambda i: (0, i, 0)),
            pl.BlockSpec((NCU, BLK), lambda i: (0, i)),
            pl.BlockSpec((H, 2 * H), lambda i: (0, 0)),
            pl.BlockSpec((H, 2 * H), lambda i: (0, 0)),
            pl.BlockSpec((2 * H, 2 * H), lambda i: (0, 0)),
            pl.BlockSpec((1, 2 * H), lambda i: (0, 0)),
        ],
        out_specs=[
            pl.BlockSpec((BLK, H), lambda i: (i, 0)),
            pl.BlockSpec((BLK, H), lambda i: (i, 0)),
        ],
        out_shape=[
            jax.ShapeDtypeStruct((N, H), jnp.float32),
            jax.ShapeDtypeStruct((N, H), jnp.float32),
        ],
    )(p_l1, agg1, cntp, W1r_a, W1r_b, W2cat, b2cat)


def _tc_c_body(p_ref, a_ref, c_ref, w_ref, b_ref, oh_ref, op_ref):
    m2 = (a_ref[0] + a_ref[1]) * _rcnt(c_ref)
    h2 = jnp.maximum(p_ref[...] + m2, 0.0)
    oh_ref[...] = h2
    op_ref[...] = (jnp.dot(h2, w_ref[...],
                           preferred_element_type=jnp.float32) + b_ref[...])


def _tc_c(p_l2b, agg2, cntp, Wmulv_l, bmulv):
    return pl.pallas_call(
        _tc_c_body,
        grid=(NBLK,),
        in_specs=[
            pl.BlockSpec((BLK, H), lambda i: (i, 0)),
            pl.BlockSpec((NCU, BLK, H), lambda i: (0, i, 0)),
            pl.BlockSpec((NCU, BLK), lambda i: (0, i)),
            pl.BlockSpec((H, 2 * H), lambda i: (0, 0)),
            pl.BlockSpec((1, 2 * H), lambda i: (0, 0)),
        ],
        out_specs=[
            pl.BlockSpec((BLK, H), lambda i: (i, 0)),
            pl.BlockSpec((BLK, 2 * H), lambda i: (i, 0)),
        ],
        out_shape=[
            jax.ShapeDtypeStruct((N, H), jnp.float32),
            jax.ShapeDtypeStruct((N, 2 * H), jnp.float32),
        ],
    )(p_l2b, agg2, cntp, Wmulv_l, bmulv)


def _tc_d_body(p_ref, a_ref, c_ref, w_ref, omu_ref, olv_ref):
    m3 = (a_ref[0] + a_ref[1]) * _rcnt(c_ref)
    res = p_ref[...] + jnp.dot(m3, w_ref[...],
                               preferred_element_type=jnp.float32)
    omu_ref[...] = res[:, :H]
    olv_ref[...] = res[:, H:]


def _tc_d(p_mulv, agg3, cntp, Wmulv_r):
    return pl.pallas_call(
        _tc_d_body,
        grid=(NBLK,),
        in_specs=[
            pl.BlockSpec((BLK, 2 * H), lambda i: (i, 0)),
            pl.BlockSpec((NCU, BLK, H), lambda i: (0, i, 0)),
            pl.BlockSpec((NCU, BLK), lambda i: (0, i)),
            pl.BlockSpec((H, 2 * H), lambda i: (0, 0)),
        ],
        out_specs=[
            pl.BlockSpec((BLK, H), lambda i: (i, 0)),
            pl.BlockSpec((BLK, H), lambda i: (i, 0)),
        ],
        out_shape=[
            jax.ShapeDtypeStruct((N, H), jnp.float32),
            jax.ShapeDtypeStruct((N, H), jnp.float32),
        ],
    )(p_mulv, agg3, cntp, Wmulv_r)


def kernel(x, edge_index, W1_l, b1, W1_r, W2_l, b2, W2_r,
           Wmu_l, bmu, Wmu_r, Wlv_l, blv, Wlv_r):
    src = edge_index[0].astype(jnp.int32)
    dst = edge_index[1].astype(jnp.int32)
    pad = E_PAD - E
    it = jnp.arange(pad, dtype=jnp.int32)
    srcs = jnp.concatenate([src, it % N]).reshape(C_TOT, CB)
    dsts = jnp.concatenate([dst, N + it % (N_ACC - N)]).reshape(C_TOT, CB)

    W2cat = jnp.concatenate([W2_l, W2_r], axis=1)
    b2cat = jnp.concatenate([b2, jnp.zeros((H,), jnp.float32)]).reshape(1, 2 * H)
    Wmulv_l = jnp.concatenate([Wmu_l, Wlv_l], axis=1)
    bmulv = jnp.concatenate([bmu, blv]).reshape(1, 2 * H)
    Wmulv_r = jnp.concatenate([Wmu_r, Wlv_r], axis=1)

    agg1, cntp = _make_sc_agg(H, True, True)(x.reshape(2 * N, H), srcs, dsts)
    p_l1 = _tc_a(x, W1_l, b1)
    p_l2b, t2 = _tc_b(p_l1, agg1, cntp, W1_r[:H], W1_r[H:], W2cat, b2cat)
    agg2 = _make_sc_agg(H, False)(t2, srcs, dsts)
    h2, p_mulv = _tc_c(p_l2b, agg2, cntp, Wmulv_l, bmulv)
    agg3 = _make_sc_agg(H, False)(h2, srcs, dsts)
    mu, lv = _tc_d(p_mulv, agg3, cntp, Wmulv_r)
    return (mu, lv)
